# Initial kernel scaffold; baseline (speedup 1.0000x reference)
import jax
import jax.numpy as jnp
from jax.experimental import pallas as pl


def kernel(x_user, x_item, edge_index_ui, edge_index_iu, Win_u, bin_u, Win_i, bin_i, l1_Wl_ui, l1_Wr_ui, l1_b_ui, l1_Wl_iu, l1_Wr_iu, l1_b_iu, l1_ln_g_u, l1_ln_b_u, l1_ln_g_i, l1_ln_b_i, l2_Wl_ui, l2_Wr_ui, l2_b_ui, l2_Wl_iu, l2_Wr_iu, l2_b_iu, l2_ln_g_u, l2_ln_b_u, l2_ln_g_i, l2_ln_b_i):
    raise NotImplementedError("write your pallas kernel here")


# speedup vs baseline: 12.4689x; 12.4689x over previous
import jax
import jax.numpy as jnp
from jax import lax
from jax.experimental import pallas as pl
from jax.experimental.pallas import tpu as pltpu
from jax.experimental.pallas import tpu_sc as plsc

N = 10000
E = 300000
D_IN = 128
H = 64
OUT = 32

NT = 16
CH = 128
NCH = 147
E_PAD = NT * NCH * CH
NP = 10240
RPT = NP // NT

_MESH = plsc.VectorSubcoreMesh(core_axis_name="c", subcore_axis_name="s")
_SC_PARAMS = pltpu.CompilerParams(use_tc_tiling_on_sc=False)


def _seg_l1(hu, hi, sui, dui, siu, diu, z64, z16, ones16,
            o_sui, o_siu, o_cui, o_ciu,
            acc, cac, sv, dv, rows, ones_v, sem):
    c = lax.axis_index("c")
    s = lax.axis_index("s")
    r0 = s * RPT
    pltpu.sync_copy(z64, acc.at[pl.ds(r0, RPT)])
    pltpu.sync_copy(z16, cac.at[pl.ds(r0, RPT)])
    pltpu.sync_copy(ones16, ones_v)

    def work(src_hbm, dst_hbm, table, o_s, o_c):
        pltpu.sync_copy(src_hbm.at[s], sv)
        pltpu.sync_copy(dst_hbm.at[s], dv)
        plsc.subcore_barrier()

        def step(j, carry):
            pltpu.async_copy(table.at[sv.at[j]], rows, sem).wait()
            pltpu.sync_copy(rows, acc.at[dv.at[j]], add=True)
            pltpu.sync_copy(ones_v, cac.at[dv.at[j]], add=True)
            return carry

        lax.fori_loop(0, NCH, step, 0)
        plsc.subcore_barrier()
        pltpu.sync_copy(acc.at[pl.ds(r0, RPT)], o_s.at[pl.ds(r0, RPT)])
        pltpu.sync_copy(cac.at[pl.ds(r0, RPT)], o_c.at[pl.ds(r0, RPT)])

    @pl.when(c == 0)
    def _():
        work(sui, dui, hu, o_sui, o_cui)

    @pl.when(c == 1)
    def _():
        work(siu, diu, hi, o_siu, o_ciu)


def _seg_l2(hu, hi, sui, dui, siu, diu, z64,
            o_sui, o_siu,
            acc, sv, dv, rows, sem):
    c = lax.axis_index("c")
    s = lax.axis_index("s")
    r0 = s * RPT
    pltpu.sync_copy(z64, acc.at[pl.ds(r0, RPT)])

    def work(src_hbm, dst_hbm, table, o_s):
        pltpu.sync_copy(src_hbm.at[s], sv)
        pltpu.sync_copy(dst_hbm.at[s], dv)
        plsc.subcore_barrier()

        def step(j, carry):
            pltpu.async_copy(table.at[sv.at[j]], rows, sem).wait()
            pltpu.sync_copy(rows, acc.at[dv.at[j]], add=True)
            return carry

        lax.fori_loop(0, NCH, step, 0)
        plsc.subcore_barrier()
        pltpu.sync_copy(acc.at[pl.ds(r0, RPT)], o_s.at[pl.ds(r0, RPT)])

    @pl.when(c == 0)
    def _():
        work(sui, dui, hu, o_sui)

    @pl.when(c == 1)
    def _():
        work(siu, diu, hi, o_siu)


def _seg_sum_l1(hu, hi, sui, dui, siu, diu):
    z64 = jnp.zeros((RPT, H), jnp.float32)
    z16 = jnp.zeros((RPT, 16), jnp.float32)
    ones16 = jnp.ones((CH, 16), jnp.float32)
    f = pl.kernel(
        _seg_l1,
        out_type=[
            jax.ShapeDtypeStruct((NP, H), jnp.float32),
            jax.ShapeDtypeStruct((NP, H), jnp.float32),
            jax.ShapeDtypeStruct((NP, 16), jnp.float32),
            jax.ShapeDtypeStruct((NP, 16), jnp.float32),
        ],
        mesh=_MESH,
        compiler_params=_SC_PARAMS,
        scratch_types=[
            pltpu.VMEM_SHARED((NP, H), jnp.float32),
            pltpu.VMEM_SHARED((NP, 16), jnp.float32),
            pltpu.VMEM((NCH, CH), jnp.int32),
            pltpu.VMEM((NCH, CH), jnp.int32),
            pltpu.VMEM((CH, H), jnp.float32),
            pltpu.VMEM((CH, 16), jnp.float32),
            pltpu.SemaphoreType.DMA,
        ],
    )
    return f(hu, hi, sui, dui, siu, diu, z64, z16, ones16)


def _seg_sum_l2(hu, hi, sui, dui, siu, diu):
    z64 = jnp.zeros((RPT, H), jnp.float32)
    f = pl.kernel(
        _seg_l2,
        out_type=[
            jax.ShapeDtypeStruct((NP, H), jnp.float32),
            jax.ShapeDtypeStruct((NP, H), jnp.float32),
        ],
        mesh=_MESH,
        compiler_params=_SC_PARAMS,
        scratch_types=[
            pltpu.VMEM_SHARED((NP, H), jnp.float32),
            pltpu.VMEM((NCH, CH), jnp.int32),
            pltpu.VMEM((NCH, CH), jnp.int32),
            pltpu.VMEM((CH, H), jnp.float32),
            pltpu.SemaphoreType.DMA,
        ],
    )
    return f(hu, hi, sui, dui, siu, diu, z64)


_BLK = 1000
_GRID = N // _BLK


def _proj_body(xu, xi, wu, wi, bu, bi, ou, oi):
    ou[...] = jnp.maximum(
        jnp.dot(xu[...], wu[...], preferred_element_type=jnp.float32)
        + bu[0:1, :], 0.0)
    oi[...] = jnp.maximum(
        jnp.dot(xi[...], wi[...], preferred_element_type=jnp.float32)
        + bi[0:1, :], 0.0)


def _proj(xu, xi, wu, bu, wi, bi):
    full = lambda shp: pl.BlockSpec(shp, lambda i: (0,) * len(shp))
    row = lambda shp: pl.BlockSpec(shp, lambda i: (i,) + (0,) * (len(shp) - 1))
    return pl.pallas_call(
        _proj_body,
        grid=(_GRID,),
        in_specs=[row((_BLK, D_IN)), row((_BLK, D_IN)),
                  full((D_IN, H)), full((D_IN, H)),
                  full((8, H)), full((8, H))],
        out_specs=[row((_BLK, H)), row((_BLK, H))],
        out_shape=[jax.ShapeDtypeStruct((N, H), jnp.float32),
                   jax.ShapeDtypeStruct((N, H), jnp.float32)],
    )(xu, xi, wu, wi, jnp.broadcast_to(bu, (8, H)), jnp.broadcast_to(bi, (8, H)))


def _combine_side(sr, cr, hr, wl, wr, b, g, be, relu):
    cnt = cr[...][:, 0:1]
    mean = sr[...] / jnp.maximum(cnt, 1.0)
    n = (jnp.dot(mean, wl[...], preferred_element_type=jnp.float32)
         + jnp.dot(hr[...], wr[...], preferred_element_type=jnp.float32)
         + b[0:1, :])
    m = jnp.mean(n, axis=-1, keepdims=True)
    v = jnp.mean((n - m) * (n - m), axis=-1, keepdims=True)
    y = (n - m) * lax.rsqrt(v + 1e-5) * g[0:1, :] + be[0:1, :]
    return jnp.maximum(y, 0.0) if relu else y


def _make_combine(dout, relu):
    def body(s_a, c_a, h_a, wl_a, wr_a, b_a, g_a, be_a,
             s_b, c_b, h_b, wl_b, wr_b, b_b, g_b, be_b, o_a, o_b):
        o_a[...] = _combine_side(s_a, c_a, h_a, wl_a, wr_a, b_a, g_a, be_a, relu)
        o_b[...] = _combine_side(s_b, c_b, h_b, wl_b, wr_b, b_b, g_b, be_b, relu)

    full = lambda shp: pl.BlockSpec(shp, lambda i: (0,) * len(shp))
    row = lambda shp: pl.BlockSpec(shp, lambda i: (i,) + (0,) * (len(shp) - 1))
    side_specs = [row((_BLK, H)), row((_BLK, 16)), row((_BLK, H)),
                  full((H, dout)), full((H, dout)),
                  full((8, dout)), full((8, dout)), full((8, dout))]

    def run(s_a, c_a, h_a, wl_a, wr_a, b_a, g_a, be_a,
            s_b, c_b, h_b, wl_b, wr_b, b_b, g_b, be_b):
        bc = lambda x: jnp.broadcast_to(x, (8, dout))
        return pl.pallas_call(
            body,
            grid=(_GRID,),
            in_specs=side_specs + side_specs,
            out_specs=[row((_BLK, dout)), row((_BLK, dout))],
            out_shape=[jax.ShapeDtypeStruct((N, dout), jnp.float32),
                       jax.ShapeDtypeStruct((N, dout), jnp.float32)],
        )(s_a, c_a, h_a, wl_a, wr_a, bc(b_a), bc(g_a), bc(be_a),
          s_b, c_b, h_b, wl_b, wr_b, bc(b_b), bc(g_b), bc(be_b))

    return run


_combine_l1 = _make_combine(H, True)
_combine_l2 = _make_combine(OUT, False)


def _prep_edges(ei):
    pad = E_PAD - E
    src = jnp.concatenate([ei[0], jnp.zeros((pad,), jnp.int32)])
    dst = jnp.concatenate([ei[1], jnp.full((pad,), N, jnp.int32)])
    return src.reshape(NT, NCH, CH), dst.reshape(NT, NCH, CH)


def kernel(x_user, x_item, edge_index_ui, edge_index_iu, Win_u, bin_u, Win_i,
           bin_i, l1_Wl_ui, l1_Wr_ui, l1_b_ui, l1_Wl_iu, l1_Wr_iu, l1_b_iu,
           l1_ln_g_u, l1_ln_b_u, l1_ln_g_i, l1_ln_b_i, l2_Wl_ui, l2_Wr_ui,
           l2_b_ui, l2_Wl_iu, l2_Wr_iu, l2_b_iu, l2_ln_g_u, l2_ln_b_u,
           l2_ln_g_i, l2_ln_b_i):
    sui, dui = _prep_edges(edge_index_ui)
    siu, diu = _prep_edges(edge_index_iu)

    h_u, h_i = _proj(x_user, x_item, Win_u, bin_u, Win_i, bin_i)

    s_ui, s_iu, c_ui, c_iu = _seg_sum_l1(h_u, h_i, sui, dui, siu, diu)
    h_i2, h_u2 = _combine_l1(
        s_ui, c_ui, h_i, l1_Wl_ui, l1_Wr_ui, l1_b_ui, l1_ln_g_i, l1_ln_b_i,
        s_iu, c_iu, h_u, l1_Wl_iu, l1_Wr_iu, l1_b_iu, l1_ln_g_u, l1_ln_b_u)

    s2_ui, s2_iu = _seg_sum_l2(h_u2, h_i2, sui, dui, siu, diu)
    out_i, out_u = _combine_l2(
        s2_ui, c_ui, h_i2, l2_Wl_ui, l2_Wr_ui, l2_b_ui, l2_ln_g_i, l2_ln_b_i,
        s2_iu, c_iu, h_u2, l2_Wl_iu, l2_Wr_iu, l2_b_iu, l2_ln_g_u, l2_ln_b_u)

    return (out_u, out_i)


# speedup vs baseline: 14.8889x; 1.1941x over previous
import jax
import jax.numpy as jnp
from jax import lax
from jax.experimental import pallas as pl
from jax.experimental.pallas import tpu as pltpu
from jax.experimental.pallas import tpu_sc as plsc

N = 10000
E = 300000
D_IN = 128
H = 64
OUT = 32

NT = 16
CH = 128
NCH = 148
E_PAD = NT * NCH * CH
NP = 10240
RPT = NP // NT

_MESH = plsc.VectorSubcoreMesh(core_axis_name="c", subcore_axis_name="s")
_SC_PARAMS = pltpu.CompilerParams(use_tc_tiling_on_sc=False)


def _pipelined_scatter(sv, dv, table, acc, rows_a, rows_b, sem_a, sem_b,
                       extra=None):
    cp_a = lambda j: pltpu.async_copy(table.at[sv.at[j]], rows_a, sem_a)
    cp_b = lambda j: pltpu.async_copy(table.at[sv.at[j]], rows_b, sem_b)
    cp_a(0)

    def step(jj, carry):
        j = jj * 2
        cp_b(j + 1)
        pltpu.make_async_copy(table.at[sv.at[j]], rows_a, sem_a).wait()
        pltpu.sync_copy(rows_a, acc.at[dv.at[j]], add=True)
        if extra is not None:
            extra(j)

        @pl.when(jj < NCH // 2 - 1)
        def _():
            cp_a(j + 2)

        pltpu.make_async_copy(table.at[sv.at[j]], rows_b, sem_b).wait()
        pltpu.sync_copy(rows_b, acc.at[dv.at[j + 1]], add=True)
        if extra is not None:
            extra(j + 1)
        return carry

    lax.fori_loop(0, NCH // 2, step, 0)


def _seg_l1(hu, hi, sui, dui, siu, diu, z64, z16, ones16,
            o_sui, o_siu, o_cui, o_ciu,
            acc, cac, sv, dv, rows_a, rows_b, ones_v, sem_a, sem_b):
    c = lax.axis_index("c")
    s = lax.axis_index("s")
    r0 = s * RPT
    pltpu.sync_copy(z64, acc.at[pl.ds(r0, RPT)])
    pltpu.sync_copy(z16, cac.at[pl.ds(r0, RPT)])
    pltpu.sync_copy(ones16, ones_v)

    def work(src_hbm, dst_hbm, table, o_s, o_c):
        pltpu.sync_copy(src_hbm.at[s], sv)
        pltpu.sync_copy(dst_hbm.at[s], dv)
        plsc.subcore_barrier()
        counts = lambda j: pltpu.sync_copy(ones_v, cac.at[dv.at[j]], add=True)
        _pipelined_scatter(sv, dv, table, acc, rows_a, rows_b, sem_a, sem_b,
                           extra=counts)
        plsc.subcore_barrier()
        pltpu.sync_copy(acc.at[pl.ds(r0, RPT)], o_s.at[pl.ds(r0, RPT)])
        pltpu.sync_copy(cac.at[pl.ds(r0, RPT)], o_c.at[pl.ds(r0, RPT)])

    @pl.when(c == 0)
    def _():
        work(sui, dui, hu, o_sui, o_cui)

    @pl.when(c == 1)
    def _():
        work(siu, diu, hi, o_siu, o_ciu)


def _seg_l2(hu, hi, sui, dui, siu, diu, z64,
            o_sui, o_siu,
            acc, sv, dv, rows_a, rows_b, sem_a, sem_b):
    c = lax.axis_index("c")
    s = lax.axis_index("s")
    r0 = s * RPT
    pltpu.sync_copy(z64, acc.at[pl.ds(r0, RPT)])

    def work(src_hbm, dst_hbm, table, o_s):
        pltpu.sync_copy(src_hbm.at[s], sv)
        pltpu.sync_copy(dst_hbm.at[s], dv)
        plsc.subcore_barrier()
        _pipelined_scatter(sv, dv, table, acc, rows_a, rows_b, sem_a, sem_b)
        plsc.subcore_barrier()
        pltpu.sync_copy(acc.at[pl.ds(r0, RPT)], o_s.at[pl.ds(r0, RPT)])

    @pl.when(c == 0)
    def _():
        work(sui, dui, hu, o_sui)

    @pl.when(c == 1)
    def _():
        work(siu, diu, hi, o_siu)


def _seg_sum_l1(hu, hi, sui, dui, siu, diu):
    z64 = jnp.zeros((RPT, H), jnp.float32)
    z16 = jnp.zeros((RPT, 16), jnp.float32)
    ones16 = jnp.ones((CH, 16), jnp.float32)
    f = pl.kernel(
        _seg_l1,
        out_type=[
            jax.ShapeDtypeStruct((NP, H), jnp.float32),
            jax.ShapeDtypeStruct((NP, H), jnp.float32),
            jax.ShapeDtypeStruct((NP, 16), jnp.float32),
            jax.ShapeDtypeStruct((NP, 16), jnp.float32),
        ],
        mesh=_MESH,
        compiler_params=_SC_PARAMS,
        scratch_types=[
            pltpu.VMEM_SHARED((NP, H), jnp.float32),
            pltpu.VMEM_SHARED((NP, 16), jnp.float32),
            pltpu.VMEM((NCH, CH), jnp.int32),
            pltpu.VMEM((NCH, CH), jnp.int32),
            pltpu.VMEM((CH, H), jnp.float32),
            pltpu.VMEM((CH, H), jnp.float32),
            pltpu.VMEM((CH, 16), jnp.float32),
            pltpu.SemaphoreType.DMA,
            pltpu.SemaphoreType.DMA,
        ],
    )
    return f(hu, hi, sui, dui, siu, diu, z64, z16, ones16)


def _seg_sum_l2(hu, hi, sui, dui, siu, diu):
    z64 = jnp.zeros((RPT, H), jnp.float32)
    f = pl.kernel(
        _seg_l2,
        out_type=[
            jax.ShapeDtypeStruct((NP, H), jnp.float32),
            jax.ShapeDtypeStruct((NP, H), jnp.float32),
        ],
        mesh=_MESH,
        compiler_params=_SC_PARAMS,
        scratch_types=[
            pltpu.VMEM_SHARED((NP, H), jnp.float32),
            pltpu.VMEM((NCH, CH), jnp.int32),
            pltpu.VMEM((NCH, CH), jnp.int32),
            pltpu.VMEM((CH, H), jnp.float32),
            pltpu.VMEM((CH, H), jnp.float32),
            pltpu.SemaphoreType.DMA,
            pltpu.SemaphoreType.DMA,
        ],
    )
    return f(hu, hi, sui, dui, siu, diu, z64)


_BLK = 1000
_GRID = N // _BLK


def _proj_body(xu, xi, wu, wi, bu, bi, ou, oi):
    ou[...] = jnp.maximum(
        jnp.dot(xu[...], wu[...], preferred_element_type=jnp.float32)
        + bu[0:1, :], 0.0)
    oi[...] = jnp.maximum(
        jnp.dot(xi[...], wi[...], preferred_element_type=jnp.float32)
        + bi[0:1, :], 0.0)


def _proj(xu, xi, wu, bu, wi, bi):
    full = lambda shp: pl.BlockSpec(shp, lambda i: (0,) * len(shp))
    row = lambda shp: pl.BlockSpec(shp, lambda i: (i,) + (0,) * (len(shp) - 1))
    return pl.pallas_call(
        _proj_body,
        grid=(_GRID,),
        in_specs=[row((_BLK, D_IN)), row((_BLK, D_IN)),
                  full((D_IN, H)), full((D_IN, H)),
                  full((8, H)), full((8, H))],
        out_specs=[row((_BLK, H)), row((_BLK, H))],
        out_shape=[jax.ShapeDtypeStruct((N, H), jnp.float32),
                   jax.ShapeDtypeStruct((N, H), jnp.float32)],
    )(xu, xi, wu, wi, jnp.broadcast_to(bu, (8, H)), jnp.broadcast_to(bi, (8, H)))


def _combine_side(sr, cr, hr, wl, wr, b, g, be, relu):
    cnt = cr[...][:, 0:1]
    mean = sr[...] / jnp.maximum(cnt, 1.0)
    n = (jnp.dot(mean, wl[...], preferred_element_type=jnp.float32)
         + jnp.dot(hr[...], wr[...], preferred_element_type=jnp.float32)
         + b[0:1, :])
    m = jnp.mean(n, axis=-1, keepdims=True)
    v = jnp.mean((n - m) * (n - m), axis=-1, keepdims=True)
    y = (n - m) * lax.rsqrt(v + 1e-5) * g[0:1, :] + be[0:1, :]
    return jnp.maximum(y, 0.0) if relu else y


def _make_combine(dout, relu):
    def body(s_a, c_a, h_a, wl_a, wr_a, b_a, g_a, be_a,
             s_b, c_b, h_b, wl_b, wr_b, b_b, g_b, be_b, o_a, o_b):
        o_a[...] = _combine_side(s_a, c_a, h_a, wl_a, wr_a, b_a, g_a, be_a, relu)
        o_b[...] = _combine_side(s_b, c_b, h_b, wl_b, wr_b, b_b, g_b, be_b, relu)

    full = lambda shp: pl.BlockSpec(shp, lambda i: (0,) * len(shp))
    row = lambda shp: pl.BlockSpec(shp, lambda i: (i,) + (0,) * (len(shp) - 1))
    side_specs = [row((_BLK, H)), row((_BLK, 16)), row((_BLK, H)),
                  full((H, dout)), full((H, dout)),
                  full((8, dout)), full((8, dout)), full((8, dout))]

    def run(s_a, c_a, h_a, wl_a, wr_a, b_a, g_a, be_a,
            s_b, c_b, h_b, wl_b, wr_b, b_b, g_b, be_b):
        bc = lambda x: jnp.broadcast_to(x, (8, dout))
        return pl.pallas_call(
            body,
            grid=(_GRID,),
            in_specs=side_specs + side_specs,
            out_specs=[row((_BLK, dout)), row((_BLK, dout))],
            out_shape=[jax.ShapeDtypeStruct((N, dout), jnp.float32),
                       jax.ShapeDtypeStruct((N, dout), jnp.float32)],
        )(s_a, c_a, h_a, wl_a, wr_a, bc(b_a), bc(g_a), bc(be_a),
          s_b, c_b, h_b, wl_b, wr_b, bc(b_b), bc(g_b), bc(be_b))

    return run


_combine_l1 = _make_combine(H, True)
_combine_l2 = _make_combine(OUT, False)


def _prep_edges(ei):
    pad = E_PAD - E
    src = jnp.concatenate([ei[0], jnp.zeros((pad,), jnp.int32)])
    dst = jnp.concatenate([ei[1], jnp.full((pad,), N, jnp.int32)])
    return src.reshape(NT, NCH, CH), dst.reshape(NT, NCH, CH)


def kernel(x_user, x_item, edge_index_ui, edge_index_iu, Win_u, bin_u, Win_i,
           bin_i, l1_Wl_ui, l1_Wr_ui, l1_b_ui, l1_Wl_iu, l1_Wr_iu, l1_b_iu,
           l1_ln_g_u, l1_ln_b_u, l1_ln_g_i, l1_ln_b_i, l2_Wl_ui, l2_Wr_ui,
           l2_b_ui, l2_Wl_iu, l2_Wr_iu, l2_b_iu, l2_ln_g_u, l2_ln_b_u,
           l2_ln_g_i, l2_ln_b_i):
    sui, dui = _prep_edges(edge_index_ui)
    siu, diu = _prep_edges(edge_index_iu)

    h_u, h_i = _proj(x_user, x_item, Win_u, bin_u, Win_i, bin_i)

    s_ui, s_iu, c_ui, c_iu = _seg_sum_l1(h_u, h_i, sui, dui, siu, diu)
    h_i2, h_u2 = _combine_l1(
        s_ui, c_ui, h_i, l1_Wl_ui, l1_Wr_ui, l1_b_ui, l1_ln_g_i, l1_ln_b_i,
        s_iu, c_iu, h_u, l1_Wl_iu, l1_Wr_iu, l1_b_iu, l1_ln_g_u, l1_ln_b_u)

    s2_ui, s2_iu = _seg_sum_l2(h_u2, h_i2, sui, dui, siu, diu)
    out_i, out_u = _combine_l2(
        s2_ui, c_ui, h_i2, l2_Wl_ui, l2_Wr_ui, l2_b_ui, l2_ln_g_i, l2_ln_b_i,
        s2_iu, c_iu, h_u2, l2_Wl_iu, l2_Wr_iu, l2_b_iu, l2_ln_g_u, l2_ln_b_u)

    return (out_u, out_i)


# speedup vs baseline: 15.1022x; 1.0143x over previous
import jax
import jax.numpy as jnp
from jax import lax
from jax.experimental import pallas as pl
from jax.experimental.pallas import tpu as pltpu
from jax.experimental.pallas import tpu_sc as plsc

N = 10000
E = 300000
D_IN = 128
H = 64
OUT = 32

NT = 16
CH = 128
NCH = 148
NB = 4
E_PAD = NT * NCH * CH
NP = 10240
RPT = NP // NT

_MESH = plsc.VectorSubcoreMesh(core_axis_name="c", subcore_axis_name="s")
_SC_PARAMS = pltpu.CompilerParams(use_tc_tiling_on_sc=False)


def _pipelined_scatter(sv, dv, table, acc, rows, gs, ss,
                       cac=None, ones_v=None, cs=None):
    W = NB // 2

    def gwait(j, k):
        pltpu.make_async_copy(table.at[sv.at[j]], rows[k], gs[k]).wait()

    def swait(k):
        pltpu.make_async_copy(rows[k], acc.at[dv.at[0]], ss[k]).wait()

    def cwait(k):
        pltpu.make_async_copy(ones_v, cac.at[dv.at[0]], cs[k]).wait()

    for k in range(W):
        pltpu.async_copy(table.at[sv.at[k]], rows[k], gs[k])

    def group(g, carry):
        j0 = g * NB
        for k in range(NB):
            j = j0 + k
            gwait(j, k)
            pltpu.async_copy(rows[k], acc.at[dv.at[j]], ss[k], add=True)
            if cac is not None:
                pltpu.async_copy(ones_v, cac.at[dv.at[j]], cs[k], add=True)
            kn = (k + W) % NB

            @pl.when(j + W < NCH)
            def _(j=j, kn=kn):
                @pl.when(j >= W)
                def _():
                    swait(kn)
                    if cac is not None:
                        cwait(kn)
                pltpu.async_copy(table.at[sv.at[j + W]], rows[kn], gs[kn])
        return carry

    lax.fori_loop(0, NCH // NB, group, 0)
    for k in range(NB):
        swait(k)
        if cac is not None:
            cwait(k)


def _seg_l1(hu, hi, sui, dui, siu, diu, z64, z16, ones16,
            o_sui, o_siu, o_cui, o_ciu,
            acc, cac, sv, dv, r0b, r1b, r2b, r3b, ones_v,
            g0, g1, g2, g3, s0, s1, s2, s3, c0, c1, c2, c3):
    c = lax.axis_index("c")
    s = lax.axis_index("s")
    r0 = s * RPT
    rows = (r0b, r1b, r2b, r3b)
    gs = (g0, g1, g2, g3)
    ss = (s0, s1, s2, s3)
    cs = (c0, c1, c2, c3)
    pltpu.sync_copy(z64, acc.at[pl.ds(r0, RPT)])
    pltpu.sync_copy(z16, cac.at[pl.ds(r0, RPT)])
    pltpu.sync_copy(ones16, ones_v)

    def work(src_hbm, dst_hbm, table, o_s, o_c):
        pltpu.sync_copy(src_hbm.at[s], sv)
        pltpu.sync_copy(dst_hbm.at[s], dv)
        plsc.subcore_barrier()
        _pipelined_scatter(sv, dv, table, acc, rows, gs, ss,
                           cac=cac, ones_v=ones_v, cs=cs)
        plsc.subcore_barrier()
        pltpu.sync_copy(acc.at[pl.ds(r0, RPT)], o_s.at[pl.ds(r0, RPT)])
        pltpu.sync_copy(cac.at[pl.ds(r0, RPT)], o_c.at[pl.ds(r0, RPT)])

    @pl.when(c == 0)
    def _():
        work(sui, dui, hu, o_sui, o_cui)

    @pl.when(c == 1)
    def _():
        work(siu, diu, hi, o_siu, o_ciu)


def _seg_l2(hu, hi, sui, dui, siu, diu, z64,
            o_sui, o_siu,
            acc, sv, dv, r0b, r1b, r2b, r3b,
            g0, g1, g2, g3, s0, s1, s2, s3):
    c = lax.axis_index("c")
    s = lax.axis_index("s")
    r0 = s * RPT
    rows = (r0b, r1b, r2b, r3b)
    gs = (g0, g1, g2, g3)
    ss = (s0, s1, s2, s3)
    pltpu.sync_copy(z64, acc.at[pl.ds(r0, RPT)])

    def work(src_hbm, dst_hbm, table, o_s):
        pltpu.sync_copy(src_hbm.at[s], sv)
        pltpu.sync_copy(dst_hbm.at[s], dv)
        plsc.subcore_barrier()
        _pipelined_scatter(sv, dv, table, acc, rows, gs, ss)
        plsc.subcore_barrier()
        pltpu.sync_copy(acc.at[pl.ds(r0, RPT)], o_s.at[pl.ds(r0, RPT)])

    @pl.when(c == 0)
    def _():
        work(sui, dui, hu, o_sui)

    @pl.when(c == 1)
    def _():
        work(siu, diu, hi, o_siu)


def _seg_sum_l1(hu, hi, sui, dui, siu, diu):
    z64 = jnp.zeros((RPT, H), jnp.float32)
    z16 = jnp.zeros((RPT, 16), jnp.float32)
    ones16 = jnp.ones((CH, 16), jnp.float32)
    f = pl.kernel(
        _seg_l1,
        out_type=[
            jax.ShapeDtypeStruct((NP, H), jnp.float32),
            jax.ShapeDtypeStruct((NP, H), jnp.float32),
            jax.ShapeDtypeStruct((NP, 16), jnp.float32),
            jax.ShapeDtypeStruct((NP, 16), jnp.float32),
        ],
        mesh=_MESH,
        compiler_params=_SC_PARAMS,
        scratch_types=[
            pltpu.VMEM_SHARED((NP, H), jnp.float32),
            pltpu.VMEM_SHARED((NP, 16), jnp.float32),
            pltpu.VMEM((NCH, CH), jnp.int32),
            pltpu.VMEM((NCH, CH), jnp.int32),
        ] + [pltpu.VMEM((CH, H), jnp.float32)] * NB + [
            pltpu.VMEM((CH, 16), jnp.float32),
        ] + [pltpu.SemaphoreType.DMA] * (3 * NB),
    )
    return f(hu, hi, sui, dui, siu, diu, z64, z16, ones16)


def _seg_sum_l2(hu, hi, sui, dui, siu, diu):
    z64 = jnp.zeros((RPT, H), jnp.float32)
    f = pl.kernel(
        _seg_l2,
        out_type=[
            jax.ShapeDtypeStruct((NP, H), jnp.float32),
            jax.ShapeDtypeStruct((NP, H), jnp.float32),
        ],
        mesh=_MESH,
        compiler_params=_SC_PARAMS,
        scratch_types=[
            pltpu.VMEM_SHARED((NP, H), jnp.float32),
            pltpu.VMEM((NCH, CH), jnp.int32),
            pltpu.VMEM((NCH, CH), jnp.int32),
        ] + [pltpu.VMEM((CH, H), jnp.float32)] * NB
          + [pltpu.SemaphoreType.DMA] * (2 * NB),
    )
    return f(hu, hi, sui, dui, siu, diu, z64)


_BLK = 1000
_GRID = N // _BLK


def _proj_body(xu, xi, wu, wi, bu, bi, ou, oi):
    ou[...] = jnp.maximum(
        jnp.dot(xu[...], wu[...], preferred_element_type=jnp.float32)
        + bu[0:1, :], 0.0)
    oi[...] = jnp.maximum(
        jnp.dot(xi[...], wi[...], preferred_element_type=jnp.float32)
        + bi[0:1, :], 0.0)


def _proj(xu, xi, wu, bu, wi, bi):
    full = lambda shp: pl.BlockSpec(shp, lambda i: (0,) * len(shp))
    row = lambda shp: pl.BlockSpec(shp, lambda i: (i,) + (0,) * (len(shp) - 1))
    return pl.pallas_call(
        _proj_body,
        grid=(_GRID,),
        in_specs=[row((_BLK, D_IN)), row((_BLK, D_IN)),
                  full((D_IN, H)), full((D_IN, H)),
                  full((8, H)), full((8, H))],
        out_specs=[row((_BLK, H)), row((_BLK, H))],
        out_shape=[jax.ShapeDtypeStruct((N, H), jnp.float32),
                   jax.ShapeDtypeStruct((N, H), jnp.float32)],
    )(xu, xi, wu, wi, jnp.broadcast_to(bu, (8, H)), jnp.broadcast_to(bi, (8, H)))


def _combine_side(sr, cr, hr, wl, wr, b, g, be, relu):
    cnt = cr[...][:, 0:1]
    mean = sr[...] / jnp.maximum(cnt, 1.0)
    n = (jnp.dot(mean, wl[...], preferred_element_type=jnp.float32)
         + jnp.dot(hr[...], wr[...], preferred_element_type=jnp.float32)
         + b[0:1, :])
    m = jnp.mean(n, axis=-1, keepdims=True)
    v = jnp.mean((n - m) * (n - m), axis=-1, keepdims=True)
    y = (n - m) * lax.rsqrt(v + 1e-5) * g[0:1, :] + be[0:1, :]
    return jnp.maximum(y, 0.0) if relu else y


def _make_combine(dout, relu):
    def body(s_a, c_a, h_a, wl_a, wr_a, b_a, g_a, be_a,
             s_b, c_b, h_b, wl_b, wr_b, b_b, g_b, be_b, o_a, o_b):
        o_a[...] = _combine_side(s_a, c_a, h_a, wl_a, wr_a, b_a, g_a, be_a, relu)
        o_b[...] = _combine_side(s_b, c_b, h_b, wl_b, wr_b, b_b, g_b, be_b, relu)

    full = lambda shp: pl.BlockSpec(shp, lambda i: (0,) * len(shp))
    row = lambda shp: pl.BlockSpec(shp, lambda i: (i,) + (0,) * (len(shp) - 1))
    side_specs = [row((_BLK, H)), row((_BLK, 16)), row((_BLK, H)),
                  full((H, dout)), full((H, dout)),
                  full((8, dout)), full((8, dout)), full((8, dout))]

    def run(s_a, c_a, h_a, wl_a, wr_a, b_a, g_a, be_a,
            s_b, c_b, h_b, wl_b, wr_b, b_b, g_b, be_b):
        bc = lambda x: jnp.broadcast_to(x, (8, dout))
        return pl.pallas_call(
            body,
            grid=(_GRID,),
            in_specs=side_specs + side_specs,
            out_specs=[row((_BLK, dout)), row((_BLK, dout))],
            out_shape=[jax.ShapeDtypeStruct((N, dout), jnp.float32),
                       jax.ShapeDtypeStruct((N, dout), jnp.float32)],
        )(s_a, c_a, h_a, wl_a, wr_a, bc(b_a), bc(g_a), bc(be_a),
          s_b, c_b, h_b, wl_b, wr_b, bc(b_b), bc(g_b), bc(be_b))

    return run


_combine_l1 = _make_combine(H, True)
_combine_l2 = _make_combine(OUT, False)


def _prep_edges(ei):
    pad = E_PAD - E
    src = jnp.concatenate([ei[0], jnp.zeros((pad,), jnp.int32)])
    dst = jnp.concatenate([ei[1], jnp.full((pad,), N, jnp.int32)])
    return src.reshape(NT, NCH, CH), dst.reshape(NT, NCH, CH)


def kernel(x_user, x_item, edge_index_ui, edge_index_iu, Win_u, bin_u, Win_i,
           bin_i, l1_Wl_ui, l1_Wr_ui, l1_b_ui, l1_Wl_iu, l1_Wr_iu, l1_b_iu,
           l1_ln_g_u, l1_ln_b_u, l1_ln_g_i, l1_ln_b_i, l2_Wl_ui, l2_Wr_ui,
           l2_b_ui, l2_Wl_iu, l2_Wr_iu, l2_b_iu, l2_ln_g_u, l2_ln_b_u,
           l2_ln_g_i, l2_ln_b_i):
    sui, dui = _prep_edges(edge_index_ui)
    siu, diu = _prep_edges(edge_index_iu)

    h_u, h_i = _proj(x_user, x_item, Win_u, bin_u, Win_i, bin_i)

    s_ui, s_iu, c_ui, c_iu = _seg_sum_l1(h_u, h_i, sui, dui, siu, diu)
    h_i2, h_u2 = _combine_l1(
        s_ui, c_ui, h_i, l1_Wl_ui, l1_Wr_ui, l1_b_ui, l1_ln_g_i, l1_ln_b_i,
        s_iu, c_iu, h_u, l1_Wl_iu, l1_Wr_iu, l1_b_iu, l1_ln_g_u, l1_ln_b_u)

    s2_ui, s2_iu = _seg_sum_l2(h_u2, h_i2, sui, dui, siu, diu)
    out_i, out_u = _combine_l2(
        s2_ui, c_ui, h_i2, l2_Wl_ui, l2_Wr_ui, l2_b_ui, l2_ln_g_i, l2_ln_b_i,
        s2_iu, c_iu, h_u2, l2_Wl_iu, l2_Wr_iu, l2_b_iu, l2_ln_g_u, l2_ln_b_u)

    return (out_u, out_i)


# speedup vs baseline: 16.3294x; 1.0813x over previous
import jax
import jax.numpy as jnp
from jax import lax
from jax.experimental import pallas as pl
from jax.experimental.pallas import tpu as pltpu
from jax.experimental.pallas import tpu_sc as plsc

N = 10000
E = 300000
D_IN = 128
H = 64
OUT = 32

NT = 16
CH = 128
NCH = 148
NB = 4
E_PAD = NT * NCH * CH
NP = 10240
RPT = NP // NT

_MESH = plsc.VectorSubcoreMesh(core_axis_name="c", subcore_axis_name="s")
_SC_PARAMS = pltpu.CompilerParams(use_tc_tiling_on_sc=False)


def _pipelined_scatter(sv, dv, table, acc, rows, gs, ss,
                       cac=None, ones_v=None, cs=None):
    W = NB // 2

    def gwait(j, k):
        pltpu.make_async_copy(table.at[sv.at[j]], rows[k], gs[k]).wait()

    def swait(k):
        pltpu.make_async_copy(rows[k], acc.at[dv.at[0]], ss[k]).wait()

    def cwait(k):
        pltpu.make_async_copy(ones_v, cac.at[dv.at[0]], cs[k]).wait()

    for k in range(W):
        pltpu.async_copy(table.at[sv.at[k]], rows[k], gs[k])

    def group(g, carry):
        j0 = g * NB
        for k in range(NB):
            j = j0 + k
            gwait(j, k)
            pltpu.async_copy(rows[k], acc.at[dv.at[j]], ss[k], add=True)
            if cac is not None:
                pltpu.async_copy(ones_v, cac.at[dv.at[j]], cs[k], add=True)
            kn = (k + W) % NB

            @pl.when(j + W < NCH)
            def _(j=j, kn=kn):
                @pl.when(j >= W)
                def _():
                    swait(kn)
                    if cac is not None:
                        cwait(kn)
                pltpu.async_copy(table.at[sv.at[j + W]], rows[kn], gs[kn])
        return carry

    lax.fori_loop(0, NCH // NB, group, 0)
    for k in range(NB):
        swait(k)
        if cac is not None:
            cwait(k)


def _cnt_body(dui, diu, z16, ones16, o_cui, o_ciu,
              cac, dv, ones_v, c0, c1, c2, c3):
    c = lax.axis_index("c")
    s = lax.axis_index("s")
    r0 = s * RPT
    cs = (c0, c1, c2, c3)
    pltpu.sync_copy(z16, cac.at[pl.ds(r0, RPT)])
    pltpu.sync_copy(ones16, ones_v)

    def cwait(k):
        pltpu.make_async_copy(ones_v, cac.at[dv.at[0]], cs[k]).wait()

    def work(dst_hbm, o_c):
        pltpu.sync_copy(dst_hbm.at[s], dv)
        plsc.subcore_barrier()

        def group(g, carry):
            for k in range(NB):
                j = g * NB + k

                @pl.when(g > 0)
                def _(k=k):
                    cwait(k)

                pltpu.async_copy(ones_v, cac.at[dv.at[j]], cs[k], add=True)
            return carry

        lax.fori_loop(0, NCH // NB, group, 0)
        for k in range(NB):
            cwait(k)
        plsc.subcore_barrier()
        pltpu.sync_copy(cac.at[pl.ds(r0, RPT)], o_c.at[pl.ds(r0, RPT)])

    @pl.when(c == 0)
    def _():
        work(dui, o_cui)

    @pl.when(c == 1)
    def _():
        work(diu, o_ciu)


def _seg_body(hu, hi, sui, dui, siu, diu, zrow,
              o_sui, o_siu,
              acc, sv, dv, r0b, r1b, r2b, r3b,
              g0, g1, g2, g3, s0, s1, s2, s3):
    c = lax.axis_index("c")
    s = lax.axis_index("s")
    r0 = s * RPT
    rows = (r0b, r1b, r2b, r3b)
    gs = (g0, g1, g2, g3)
    ss = (s0, s1, s2, s3)
    pltpu.sync_copy(zrow, acc.at[pl.ds(r0, RPT)])

    def work(src_hbm, dst_hbm, table, o_s):
        pltpu.sync_copy(src_hbm.at[s], sv)
        pltpu.sync_copy(dst_hbm.at[s], dv)
        plsc.subcore_barrier()
        _pipelined_scatter(sv, dv, table, acc, rows, gs, ss)
        plsc.subcore_barrier()
        pltpu.sync_copy(acc.at[pl.ds(r0, RPT)], o_s.at[pl.ds(r0, RPT)])

    @pl.when(c == 0)
    def _():
        work(sui, dui, hu, o_sui)

    @pl.when(c == 1)
    def _():
        work(siu, diu, hi, o_siu)


def _seg_cnt(dui, diu):
    z16 = jnp.zeros((RPT, 16), jnp.float32)
    ones16 = jnp.ones((CH, 16), jnp.float32)
    f = pl.kernel(
        _cnt_body,
        out_type=[
            jax.ShapeDtypeStruct((NP, 16), jnp.float32),
            jax.ShapeDtypeStruct((NP, 16), jnp.float32),
        ],
        mesh=_MESH,
        compiler_params=_SC_PARAMS,
        scratch_types=[
            pltpu.VMEM_SHARED((NP, 16), jnp.float32),
            pltpu.VMEM((NCH, CH), jnp.int32),
            pltpu.VMEM((CH, 16), jnp.float32),
        ] + [pltpu.SemaphoreType.DMA] * NB,
    )
    return f(dui, diu, z16, ones16)


def _seg_sum(hu, hi, sui, dui, siu, diu, feat):
    zrow = jnp.zeros((RPT, feat), jnp.float32)
    f = pl.kernel(
        _seg_body,
        out_type=[
            jax.ShapeDtypeStruct((NP, feat), jnp.float32),
            jax.ShapeDtypeStruct((NP, feat), jnp.float32),
        ],
        mesh=_MESH,
        compiler_params=_SC_PARAMS,
        scratch_types=[
            pltpu.VMEM_SHARED((NP, feat), jnp.float32),
            pltpu.VMEM((NCH, CH), jnp.int32),
            pltpu.VMEM((NCH, CH), jnp.int32),
        ] + [pltpu.VMEM((CH, feat), jnp.float32)] * NB
          + [pltpu.SemaphoreType.DMA] * (2 * NB),
    )
    return f(hu, hi, sui, dui, siu, diu, zrow)


_BLK = 1000
_GRID = N // _BLK


def _proj_body(xu, xi, wu, wi, bu, bi, ou, oi):
    ou[...] = jnp.maximum(
        jnp.dot(xu[...], wu[...], preferred_element_type=jnp.float32)
        + bu[0:1, :], 0.0)
    oi[...] = jnp.maximum(
        jnp.dot(xi[...], wi[...], preferred_element_type=jnp.float32)
        + bi[0:1, :], 0.0)


def _proj(xu, xi, wu, bu, wi, bi):
    full = lambda shp: pl.BlockSpec(shp, lambda i: (0,) * len(shp))
    row = lambda shp: pl.BlockSpec(shp, lambda i: (i,) + (0,) * (len(shp) - 1))
    return pl.pallas_call(
        _proj_body,
        grid=(_GRID,),
        in_specs=[row((_BLK, D_IN)), row((_BLK, D_IN)),
                  full((D_IN, H)), full((D_IN, H)),
                  full((8, H)), full((8, H))],
        out_specs=[row((_BLK, H)), row((_BLK, H))],
        out_shape=[jax.ShapeDtypeStruct((N, H), jnp.float32),
                   jax.ShapeDtypeStruct((N, H), jnp.float32)],
    )(xu, xi, wu, wi, jnp.broadcast_to(bu, (8, H)), jnp.broadcast_to(bi, (8, H)))


_full = lambda shp: pl.BlockSpec(shp, lambda i: (0,) * len(shp))
_row = lambda shp: pl.BlockSpec(shp, lambda i: (i,) + (0,) * (len(shp) - 1))


def _ln_act(n, g, be, relu):
    m = jnp.mean(n, axis=-1, keepdims=True)
    v = jnp.mean((n - m) * (n - m), axis=-1, keepdims=True)
    y = (n - m) * lax.rsqrt(v + 1e-5) * g[0:1, :] + be[0:1, :]
    return jnp.maximum(y, 0.0) if relu else y


def _c1_side(sr, cr, hr, wl, wr, b, g, be, wn, o, op):
    cnt = cr[...][:, 0:1]
    mean = sr[...] / jnp.maximum(cnt, 1.0)
    n = (jnp.dot(mean, wl[...], preferred_element_type=jnp.float32)
         + jnp.dot(hr[...], wr[...], preferred_element_type=jnp.float32)
         + b[0:1, :])
    y = _ln_act(n, g, be, True)
    o[...] = y
    op[...] = jnp.dot(y, wn[...], preferred_element_type=jnp.float32)


def _combine1_body(s_a, c_a, h_a, wl_a, wr_a, b_a, g_a, be_a, wn_a,
                   s_b, c_b, h_b, wl_b, wr_b, b_b, g_b, be_b, wn_b,
                   o_a, op_a, o_b, op_b):
    _c1_side(s_a, c_a, h_a, wl_a, wr_a, b_a, g_a, be_a, wn_a, o_a, op_a)
    _c1_side(s_b, c_b, h_b, wl_b, wr_b, b_b, g_b, be_b, wn_b, o_b, op_b)


def _combine_l1(s_a, c_a, h_a, wl_a, wr_a, b_a, g_a, be_a, wn_a,
                s_b, c_b, h_b, wl_b, wr_b, b_b, g_b, be_b, wn_b):
    bc = lambda x: jnp.broadcast_to(x, (8, H))
    side = [_row((_BLK, H)), _row((_BLK, 16)), _row((_BLK, H)),
            _full((H, H)), _full((H, H)),
            _full((8, H)), _full((8, H)), _full((8, H)), _full((H, OUT))]
    return pl.pallas_call(
        _combine1_body,
        grid=(_GRID,),
        in_specs=side + side,
        out_specs=[_row((_BLK, H)), _row((_BLK, OUT)),
                   _row((_BLK, H)), _row((_BLK, OUT))],
        out_shape=[jax.ShapeDtypeStruct((N, H), jnp.float32),
                   jax.ShapeDtypeStruct((N, OUT), jnp.float32),
                   jax.ShapeDtypeStruct((N, H), jnp.float32),
                   jax.ShapeDtypeStruct((N, OUT), jnp.float32)],
    )(s_a, c_a, h_a, wl_a, wr_a, bc(b_a), bc(g_a), bc(be_a), wn_a,
      s_b, c_b, h_b, wl_b, wr_b, bc(b_b), bc(g_b), bc(be_b), wn_b)


def _c2_side(sr, cr, hr, wr, b, g, be, o):
    cnt = cr[...][:, 0:1]
    n = (sr[...] / jnp.maximum(cnt, 1.0)
         + jnp.dot(hr[...], wr[...], preferred_element_type=jnp.float32)
         + b[0:1, :])
    o[...] = _ln_act(n, g, be, False)


def _combine2_body(s_a, c_a, h_a, wr_a, b_a, g_a, be_a,
                   s_b, c_b, h_b, wr_b, b_b, g_b, be_b, o_a, o_b):
    _c2_side(s_a, c_a, h_a, wr_a, b_a, g_a, be_a, o_a)
    _c2_side(s_b, c_b, h_b, wr_b, b_b, g_b, be_b, o_b)


def _combine_l2(s_a, c_a, h_a, wr_a, b_a, g_a, be_a,
                s_b, c_b, h_b, wr_b, b_b, g_b, be_b):
    bc = lambda x: jnp.broadcast_to(x, (8, OUT))
    side = [_row((_BLK, OUT)), _row((_BLK, 16)), _row((_BLK, H)),
            _full((H, OUT)),
            _full((8, OUT)), _full((8, OUT)), _full((8, OUT))]
    return pl.pallas_call(
        _combine2_body,
        grid=(_GRID,),
        in_specs=side + side,
        out_specs=[_row((_BLK, OUT)), _row((_BLK, OUT))],
        out_shape=[jax.ShapeDtypeStruct((N, OUT), jnp.float32),
                   jax.ShapeDtypeStruct((N, OUT), jnp.float32)],
    )(s_a, c_a, h_a, wr_a, bc(b_a), bc(g_a), bc(be_a),
      s_b, c_b, h_b, wr_b, bc(b_b), bc(g_b), bc(be_b))


def _prep_edges(ei):
    pad = E_PAD - E
    src = jnp.concatenate([ei[0], jnp.zeros((pad,), jnp.int32)])
    dst = jnp.concatenate([ei[1], jnp.full((pad,), N, jnp.int32)])
    return src.reshape(NT, NCH, CH), dst.reshape(NT, NCH, CH)


def kernel(x_user, x_item, edge_index_ui, edge_index_iu, Win_u, bin_u, Win_i,
           bin_i, l1_Wl_ui, l1_Wr_ui, l1_b_ui, l1_Wl_iu, l1_Wr_iu, l1_b_iu,
           l1_ln_g_u, l1_ln_b_u, l1_ln_g_i, l1_ln_b_i, l2_Wl_ui, l2_Wr_ui,
           l2_b_ui, l2_Wl_iu, l2_Wr_iu, l2_b_iu, l2_ln_g_u, l2_ln_b_u,
           l2_ln_g_i, l2_ln_b_i):
    sui, dui = _prep_edges(edge_index_ui)
    siu, diu = _prep_edges(edge_index_iu)

    c_ui, c_iu = _seg_cnt(dui, diu)
    h_u, h_i = _proj(x_user, x_item, Win_u, bin_u, Win_i, bin_i)

    s_ui, s_iu = _seg_sum(h_u, h_i, sui, dui, siu, diu, H)
    h_i2, hp_i2, h_u2, hp_u2 = _combine_l1(
        s_ui, c_ui, h_i, l1_Wl_ui, l1_Wr_ui, l1_b_ui, l1_ln_g_i, l1_ln_b_i,
        l2_Wl_iu,
        s_iu, c_iu, h_u, l1_Wl_iu, l1_Wr_iu, l1_b_iu, l1_ln_g_u, l1_ln_b_u,
        l2_Wl_ui)

    s2_ui, s2_iu = _seg_sum(hp_u2, hp_i2, sui, dui, siu, diu, OUT)
    out_i, out_u = _combine_l2(
        s2_ui, c_ui, h_i2, l2_Wr_ui, l2_b_ui, l2_ln_g_i, l2_ln_b_i,
        s2_iu, c_iu, h_u2, l2_Wr_iu, l2_b_iu, l2_ln_g_u, l2_ln_b_u)

    return (out_u, out_i)


# speedup vs baseline: 17.2303x; 1.0552x over previous
import jax
import jax.numpy as jnp
from jax import lax
from jax.experimental import pallas as pl
from jax.experimental.pallas import tpu as pltpu
from jax.experimental.pallas import tpu_sc as plsc

N = 10000
E = 300000
D_IN = 128
H = 64
OUT = 32

NT = 16
CH = 128
NCH = 148
NB = 4
E_PAD = NT * NCH * CH
NP = 10240
RPT = NP // NT

_MESH = plsc.VectorSubcoreMesh(core_axis_name="c", subcore_axis_name="s")
_SC_PARAMS = pltpu.CompilerParams(use_tc_tiling_on_sc=False)


def _pipelined_scatter(sv, dv, table, acc, rows, gs, ss,
                       cac=None, ones_v=None, cs=None):
    W = NB // 2

    def gwait(j, k):
        pltpu.make_async_copy(table.at[sv.at[j]], rows[k], gs[k]).wait()

    def swait(k):
        pltpu.make_async_copy(rows[k], acc.at[dv.at[0]], ss[k]).wait()

    def cwait(k):
        pltpu.make_async_copy(ones_v, cac.at[dv.at[0]], cs[k]).wait()

    for k in range(W):
        pltpu.async_copy(table.at[sv.at[k]], rows[k], gs[k])

    def group(g, carry):
        j0 = g * NB
        for k in range(NB):
            j = j0 + k
            gwait(j, k)
            pltpu.async_copy(rows[k], acc.at[dv.at[j]], ss[k], add=True)
            if cac is not None:
                pltpu.async_copy(ones_v, cac.at[dv.at[j]], cs[k], add=True)
            kn = (k + W) % NB

            @pl.when(j + W < NCH)
            def _(j=j, kn=kn):
                @pl.when(j >= W)
                def _():
                    swait(kn)
                    if cac is not None:
                        cwait(kn)
                pltpu.async_copy(table.at[sv.at[j + W]], rows[kn], gs[kn])
        return carry

    lax.fori_loop(0, NCH // NB, group, 0)
    for k in range(NB):
        swait(k)
        if cac is not None:
            cwait(k)


def _seg_body_cnt(hu, hi, sui, dui, siu, diu, zrow, z16, ones16,
                  o_sui, o_siu, o_cui, o_ciu,
                  acc, cac, sv, dv, r0b, r1b, r2b, r3b, ones_v,
                  g0, g1, g2, g3, s0, s1, s2, s3, c0, c1, c2, c3):
    c = lax.axis_index("c")
    s = lax.axis_index("s")
    r0 = s * RPT
    rows = (r0b, r1b, r2b, r3b)
    gs = (g0, g1, g2, g3)
    ss = (s0, s1, s2, s3)
    cs = (c0, c1, c2, c3)
    pltpu.sync_copy(zrow, acc.at[pl.ds(r0, RPT)])
    pltpu.sync_copy(z16, cac.at[pl.ds(r0, RPT)])
    pltpu.sync_copy(ones16, ones_v)

    def work(src_hbm, dst_hbm, table, o_s, o_c):
        pltpu.sync_copy(src_hbm.at[s], sv)
        pltpu.sync_copy(dst_hbm.at[s], dv)
        plsc.subcore_barrier()
        _pipelined_scatter(sv, dv, table, acc, rows, gs, ss,
                           cac=cac, ones_v=ones_v, cs=cs)
        plsc.subcore_barrier()
        pltpu.sync_copy(acc.at[pl.ds(r0, RPT)], o_s.at[pl.ds(r0, RPT)])
        pltpu.sync_copy(cac.at[pl.ds(r0, RPT)], o_c.at[pl.ds(r0, RPT)])

    @pl.when(c == 0)
    def _():
        work(sui, dui, hu, o_sui, o_cui)

    @pl.when(c == 1)
    def _():
        work(siu, diu, hi, o_siu, o_ciu)


def _seg_body(hu, hi, sui, dui, siu, diu, zrow,
              o_sui, o_siu,
              acc, sv, dv, r0b, r1b, r2b, r3b,
              g0, g1, g2, g3, s0, s1, s2, s3):
    c = lax.axis_index("c")
    s = lax.axis_index("s")
    r0 = s * RPT
    rows = (r0b, r1b, r2b, r3b)
    gs = (g0, g1, g2, g3)
    ss = (s0, s1, s2, s3)
    pltpu.sync_copy(zrow, acc.at[pl.ds(r0, RPT)])

    def work(src_hbm, dst_hbm, table, o_s):
        pltpu.sync_copy(src_hbm.at[s], sv)
        pltpu.sync_copy(dst_hbm.at[s], dv)
        plsc.subcore_barrier()
        _pipelined_scatter(sv, dv, table, acc, rows, gs, ss)
        plsc.subcore_barrier()
        pltpu.sync_copy(acc.at[pl.ds(r0, RPT)], o_s.at[pl.ds(r0, RPT)])

    @pl.when(c == 0)
    def _():
        work(sui, dui, hu, o_sui)

    @pl.when(c == 1)
    def _():
        work(siu, diu, hi, o_siu)


def _seg_sum_cnt(hu, hi, sui, dui, siu, diu):
    zrow = jnp.zeros((RPT, H), jnp.float32)
    z16 = jnp.zeros((RPT, 16), jnp.float32)
    ones16 = jnp.ones((CH, 16), jnp.float32)
    f = pl.kernel(
        _seg_body_cnt,
        out_type=[
            jax.ShapeDtypeStruct((NP, H), jnp.float32),
            jax.ShapeDtypeStruct((NP, H), jnp.float32),
            jax.ShapeDtypeStruct((NP, 16), jnp.float32),
            jax.ShapeDtypeStruct((NP, 16), jnp.float32),
        ],
        mesh=_MESH,
        compiler_params=_SC_PARAMS,
        scratch_types=[
            pltpu.VMEM_SHARED((NP, H), jnp.float32),
            pltpu.VMEM_SHARED((NP, 16), jnp.float32),
            pltpu.VMEM((NCH, CH), jnp.int32),
            pltpu.VMEM((NCH, CH), jnp.int32),
        ] + [pltpu.VMEM((CH, H), jnp.float32)] * NB + [
            pltpu.VMEM((CH, 16), jnp.float32),
        ] + [pltpu.SemaphoreType.DMA] * (3 * NB),
    )
    return f(hu, hi, sui, dui, siu, diu, zrow, z16, ones16)


def _seg_sum(hu, hi, sui, dui, siu, diu, feat):
    zrow = jnp.zeros((RPT, feat), jnp.float32)
    f = pl.kernel(
        _seg_body,
        out_type=[
            jax.ShapeDtypeStruct((NP, feat), jnp.float32),
            jax.ShapeDtypeStruct((NP, feat), jnp.float32),
        ],
        mesh=_MESH,
        compiler_params=_SC_PARAMS,
        scratch_types=[
            pltpu.VMEM_SHARED((NP, feat), jnp.float32),
            pltpu.VMEM((NCH, CH), jnp.int32),
            pltpu.VMEM((NCH, CH), jnp.int32),
        ] + [pltpu.VMEM((CH, feat), jnp.float32)] * NB
          + [pltpu.SemaphoreType.DMA] * (2 * NB),
    )
    return f(hu, hi, sui, dui, siu, diu, zrow)


_BLK = 1000
_GRID = N // _BLK


def _proj_body(xu, xi, wu, wi, bu, bi, ou, oi):
    ou[...] = jnp.maximum(
        jnp.dot(xu[...], wu[...], preferred_element_type=jnp.float32)
        + bu[0:1, :], 0.0)
    oi[...] = jnp.maximum(
        jnp.dot(xi[...], wi[...], preferred_element_type=jnp.float32)
        + bi[0:1, :], 0.0)


def _proj(xu, xi, wu, bu, wi, bi):
    full = lambda shp: pl.BlockSpec(shp, lambda i: (0,) * len(shp))
    row = lambda shp: pl.BlockSpec(shp, lambda i: (i,) + (0,) * (len(shp) - 1))
    return pl.pallas_call(
        _proj_body,
        grid=(_GRID,),
        in_specs=[row((_BLK, D_IN)), row((_BLK, D_IN)),
                  full((D_IN, H)), full((D_IN, H)),
                  full((8, H)), full((8, H))],
        out_specs=[row((_BLK, H)), row((_BLK, H))],
        out_shape=[jax.ShapeDtypeStruct((N, H), jnp.float32),
                   jax.ShapeDtypeStruct((N, H), jnp.float32)],
    )(xu, xi, wu, wi, jnp.broadcast_to(bu, (8, H)), jnp.broadcast_to(bi, (8, H)))


_full = lambda shp: pl.BlockSpec(shp, lambda i: (0,) * len(shp))
_row = lambda shp: pl.BlockSpec(shp, lambda i: (i,) + (0,) * (len(shp) - 1))


def _ln_act(n, g, be, relu):
    m = jnp.mean(n, axis=-1, keepdims=True)
    v = jnp.mean((n - m) * (n - m), axis=-1, keepdims=True)
    y = (n - m) * lax.rsqrt(v + 1e-5) * g[0:1, :] + be[0:1, :]
    return jnp.maximum(y, 0.0) if relu else y


def _c1_side(sr, cr, hr, wl, wr, b, g, be, wn, o, op):
    cnt = cr[...][:, 0:1]
    mean = sr[...] / jnp.maximum(cnt, 1.0)
    n = (jnp.dot(mean, wl[...], preferred_element_type=jnp.float32)
         + jnp.dot(hr[...], wr[...], preferred_element_type=jnp.float32)
         + b[0:1, :])
    y = _ln_act(n, g, be, True)
    o[...] = y
    op[...] = jnp.dot(y, wn[...], preferred_element_type=jnp.float32)


def _combine1_body(s_a, c_a, h_a, wl_a, wr_a, b_a, g_a, be_a, wn_a,
                   s_b, c_b, h_b, wl_b, wr_b, b_b, g_b, be_b, wn_b,
                   o_a, op_a, o_b, op_b):
    _c1_side(s_a, c_a, h_a, wl_a, wr_a, b_a, g_a, be_a, wn_a, o_a, op_a)
    _c1_side(s_b, c_b, h_b, wl_b, wr_b, b_b, g_b, be_b, wn_b, o_b, op_b)


def _combine_l1(s_a, c_a, h_a, wl_a, wr_a, b_a, g_a, be_a, wn_a,
                s_b, c_b, h_b, wl_b, wr_b, b_b, g_b, be_b, wn_b):
    bc = lambda x: jnp.broadcast_to(x, (8, H))
    side = [_row((_BLK, H)), _row((_BLK, 16)), _row((_BLK, H)),
            _full((H, H)), _full((H, H)),
            _full((8, H)), _full((8, H)), _full((8, H)), _full((H, OUT))]
    return pl.pallas_call(
        _combine1_body,
        grid=(_GRID,),
        in_specs=side + side,
        out_specs=[_row((_BLK, H)), _row((_BLK, OUT)),
                   _row((_BLK, H)), _row((_BLK, OUT))],
        out_shape=[jax.ShapeDtypeStruct((N, H), jnp.float32),
                   jax.ShapeDtypeStruct((N, OUT), jnp.float32),
                   jax.ShapeDtypeStruct((N, H), jnp.float32),
                   jax.ShapeDtypeStruct((N, OUT), jnp.float32)],
    )(s_a, c_a, h_a, wl_a, wr_a, bc(b_a), bc(g_a), bc(be_a), wn_a,
      s_b, c_b, h_b, wl_b, wr_b, bc(b_b), bc(g_b), bc(be_b), wn_b)


def _c2_side(sr, cr, hr, wr, b, g, be, o):
    cnt = cr[...][:, 0:1]
    n = (sr[...] / jnp.maximum(cnt, 1.0)
         + jnp.dot(hr[...], wr[...], preferred_element_type=jnp.float32)
         + b[0:1, :])
    o[...] = _ln_act(n, g, be, False)


def _combine2_body(s_a, c_a, h_a, wr_a, b_a, g_a, be_a,
                   s_b, c_b, h_b, wr_b, b_b, g_b, be_b, o_a, o_b):
    _c2_side(s_a, c_a, h_a, wr_a, b_a, g_a, be_a, o_a)
    _c2_side(s_b, c_b, h_b, wr_b, b_b, g_b, be_b, o_b)


def _combine_l2(s_a, c_a, h_a, wr_a, b_a, g_a, be_a,
                s_b, c_b, h_b, wr_b, b_b, g_b, be_b):
    bc = lambda x: jnp.broadcast_to(x, (8, OUT))
    side = [_row((_BLK, OUT)), _row((_BLK, 16)), _row((_BLK, H)),
            _full((H, OUT)),
            _full((8, OUT)), _full((8, OUT)), _full((8, OUT))]
    return pl.pallas_call(
        _combine2_body,
        grid=(_GRID,),
        in_specs=side + side,
        out_specs=[_row((_BLK, OUT)), _row((_BLK, OUT))],
        out_shape=[jax.ShapeDtypeStruct((N, OUT), jnp.float32),
                   jax.ShapeDtypeStruct((N, OUT), jnp.float32)],
    )(s_a, c_a, h_a, wr_a, bc(b_a), bc(g_a), bc(be_a),
      s_b, c_b, h_b, wr_b, bc(b_b), bc(g_b), bc(be_b))


def _prep_edges(ei):
    pad = E_PAD - E
    src = jnp.concatenate([ei[0], jnp.zeros((pad,), jnp.int32)])
    dst = jnp.concatenate([ei[1], jnp.full((pad,), N, jnp.int32)])
    return src.reshape(NT, NCH, CH), dst.reshape(NT, NCH, CH)


def kernel(x_user, x_item, edge_index_ui, edge_index_iu, Win_u, bin_u, Win_i,
           bin_i, l1_Wl_ui, l1_Wr_ui, l1_b_ui, l1_Wl_iu, l1_Wr_iu, l1_b_iu,
           l1_ln_g_u, l1_ln_b_u, l1_ln_g_i, l1_ln_b_i, l2_Wl_ui, l2_Wr_ui,
           l2_b_ui, l2_Wl_iu, l2_Wr_iu, l2_b_iu, l2_ln_g_u, l2_ln_b_u,
           l2_ln_g_i, l2_ln_b_i):
    sui, dui = _prep_edges(edge_index_ui)
    siu, diu = _prep_edges(edge_index_iu)

    h_u, h_i = _proj(x_user, x_item, Win_u, bin_u, Win_i, bin_i)

    s_ui, s_iu, c_ui, c_iu = _seg_sum_cnt(h_u, h_i, sui, dui, siu, diu)
    h_i2, hp_i2, h_u2, hp_u2 = _combine_l1(
        s_ui, c_ui, h_i, l1_Wl_ui, l1_Wr_ui, l1_b_ui, l1_ln_g_i, l1_ln_b_i,
        l2_Wl_iu,
        s_iu, c_iu, h_u, l1_Wl_iu, l1_Wr_iu, l1_b_iu, l1_ln_g_u, l1_ln_b_u,
        l2_Wl_ui)

    s2_ui, s2_iu = _seg_sum(hp_u2, hp_i2, sui, dui, siu, diu, OUT)
    out_i, out_u = _combine_l2(
        s2_ui, c_ui, h_i2, l2_Wr_ui, l2_b_ui, l2_ln_g_i, l2_ln_b_i,
        s2_iu, c_iu, h_u2, l2_Wr_iu, l2_b_iu, l2_ln_g_u, l2_ln_b_u)

    return (out_u, out_i)


# speedup vs baseline: 17.7538x; 1.0304x over previous
import jax
import jax.numpy as jnp
from jax import lax
from jax.experimental import pallas as pl
from jax.experimental.pallas import tpu as pltpu
from jax.experimental.pallas import tpu_sc as plsc

N = 10000
E = 300000
D_IN = 128
H = 64
OUT = 32

NT = 16
CH = 128
NCH = 148
NB = 4
E_PAD = NT * NCH * CH
NP = 10240
RPT = NP // NT

_MESH = plsc.VectorSubcoreMesh(core_axis_name="c", subcore_axis_name="s")
_SC_PARAMS = pltpu.CompilerParams(use_tc_tiling_on_sc=False)


def _pipelined_scatter(sv, dv, table, acc, rows, gs, ss,
                       cac=None, ones_v=None, cs=None):
    W = NB // 2

    def gwait(j, k):
        pltpu.make_async_copy(table.at[sv.at[j]], rows[k], gs[k]).wait()

    def swait(k):
        pltpu.make_async_copy(rows[k], acc.at[dv.at[0]], ss[k]).wait()

    def cwait(k):
        pltpu.make_async_copy(ones_v, cac.at[dv.at[0]], cs[k]).wait()

    for k in range(W):
        pltpu.async_copy(table.at[sv.at[k]], rows[k], gs[k])

    def group(g, carry):
        j0 = g * NB
        for k in range(NB):
            j = j0 + k
            gwait(j, k)
            pltpu.async_copy(rows[k], acc.at[dv.at[j]], ss[k], add=True)
            if cac is not None:
                pltpu.async_copy(ones_v, cac.at[dv.at[j]], cs[k], add=True)
            kn = (k + W) % NB

            @pl.when(j + W < NCH)
            def _(j=j, kn=kn):
                @pl.when(j >= W)
                def _():
                    swait(kn)
                    if cac is not None:
                        cwait(kn)
                pltpu.async_copy(table.at[sv.at[j + W]], rows[kn], gs[kn])
        return carry

    lax.fori_loop(0, NCH // NB, group, 0)
    for k in range(NB):
        swait(k)
        if cac is not None:
            cwait(k)


def _seg_body_cnt(hu, hi, sui, dui, siu, diu, zrow, z16, ones16,
                  o_sui, o_siu, o_cui, o_ciu,
                  acc, cac, sv, dv, r0b, r1b, r2b, r3b, ones_v,
                  g0, g1, g2, g3, s0, s1, s2, s3, c0, c1, c2, c3):
    c = lax.axis_index("c")
    s = lax.axis_index("s")
    r0 = s * RPT
    rows = (r0b, r1b, r2b, r3b)
    gs = (g0, g1, g2, g3)
    ss = (s0, s1, s2, s3)
    cs = (c0, c1, c2, c3)
    pltpu.sync_copy(zrow, acc.at[pl.ds(r0, RPT)])
    pltpu.sync_copy(z16, cac.at[pl.ds(r0, RPT)])
    pltpu.sync_copy(ones16, ones_v)

    def work(src_hbm, dst_hbm, table, o_s, o_c):
        pltpu.sync_copy(src_hbm.at[s], sv)
        pltpu.sync_copy(dst_hbm.at[s], dv)
        plsc.subcore_barrier()
        _pipelined_scatter(sv, dv, table, acc, rows, gs, ss,
                           cac=cac, ones_v=ones_v, cs=cs)
        plsc.subcore_barrier()
        pltpu.sync_copy(acc.at[pl.ds(r0, RPT)], o_s.at[pl.ds(r0, RPT)])
        pltpu.sync_copy(cac.at[pl.ds(r0, RPT)], o_c.at[pl.ds(r0, RPT)])

    @pl.when(c == 0)
    def _():
        work(sui, dui, hu, o_sui, o_cui)

    @pl.when(c == 1)
    def _():
        work(siu, diu, hi, o_siu, o_ciu)


def _seg_body(hu, hi, sui, dui, siu, diu, zrow,
              o_sui, o_siu,
              acc, sv, dv, r0b, r1b, r2b, r3b,
              g0, g1, g2, g3, s0, s1, s2, s3):
    c = lax.axis_index("c")
    s = lax.axis_index("s")
    r0 = s * RPT
    rows = (r0b, r1b, r2b, r3b)
    gs = (g0, g1, g2, g3)
    ss = (s0, s1, s2, s3)
    pltpu.sync_copy(zrow, acc.at[pl.ds(r0, RPT)])

    def work(src_hbm, dst_hbm, table, o_s):
        pltpu.sync_copy(src_hbm.at[s], sv)
        pltpu.sync_copy(dst_hbm.at[s], dv)
        plsc.subcore_barrier()
        _pipelined_scatter(sv, dv, table, acc, rows, gs, ss)
        plsc.subcore_barrier()
        pltpu.sync_copy(acc.at[pl.ds(r0, RPT)], o_s.at[pl.ds(r0, RPT)])

    @pl.when(c == 0)
    def _():
        work(sui, dui, hu, o_sui)

    @pl.when(c == 1)
    def _():
        work(siu, diu, hi, o_siu)


def _seg_sum_cnt(hu, hi, sui, dui, siu, diu):
    zrow = jnp.zeros((RPT, H), jnp.float32)
    z16 = jnp.zeros((RPT, 16), jnp.float32)
    ones16 = jnp.ones((CH, 16), jnp.float32)
    f = pl.kernel(
        _seg_body_cnt,
        out_type=[
            jax.ShapeDtypeStruct((NP, H), jnp.float32),
            jax.ShapeDtypeStruct((NP, H), jnp.float32),
            jax.ShapeDtypeStruct((NP, 16), jnp.float32),
            jax.ShapeDtypeStruct((NP, 16), jnp.float32),
        ],
        mesh=_MESH,
        compiler_params=_SC_PARAMS,
        scratch_types=[
            pltpu.VMEM_SHARED((NP, H), jnp.float32),
            pltpu.VMEM_SHARED((NP, 16), jnp.float32),
            pltpu.VMEM((NCH, CH), jnp.int32),
            pltpu.VMEM((NCH, CH), jnp.int32),
        ] + [pltpu.VMEM((CH, H), jnp.float32)] * NB + [
            pltpu.VMEM((CH, 16), jnp.float32),
        ] + [pltpu.SemaphoreType.DMA] * (3 * NB),
    )
    return f(hu, hi, sui, dui, siu, diu, zrow, z16, ones16)


def _seg_sum(hu, hi, sui, dui, siu, diu, feat):
    zrow = jnp.zeros((RPT, feat), jnp.float32)
    f = pl.kernel(
        _seg_body,
        out_type=[
            jax.ShapeDtypeStruct((NP, feat), jnp.float32),
            jax.ShapeDtypeStruct((NP, feat), jnp.float32),
        ],
        mesh=_MESH,
        compiler_params=_SC_PARAMS,
        scratch_types=[
            pltpu.VMEM_SHARED((NP, feat), jnp.float32),
            pltpu.VMEM((NCH, CH), jnp.int32),
            pltpu.VMEM((NCH, CH), jnp.int32),
        ] + [pltpu.VMEM((CH, feat), jnp.float32)] * NB
          + [pltpu.SemaphoreType.DMA] * (2 * NB),
    )
    return f(hu, hi, sui, dui, siu, diu, zrow)


_BLK = 2000
_GRID = N // _BLK


def _proj_body(xu, xi, wu, wi, bu, bi, ou, oi):
    ou[...] = jnp.maximum(
        jnp.dot(xu[...], wu[...], preferred_element_type=jnp.float32)
        + bu[0:1, :], 0.0)
    oi[...] = jnp.maximum(
        jnp.dot(xi[...], wi[...], preferred_element_type=jnp.float32)
        + bi[0:1, :], 0.0)


def _proj(xu, xi, wu, bu, wi, bi):
    full = lambda shp: pl.BlockSpec(shp, lambda i: (0,) * len(shp))
    row = lambda shp: pl.BlockSpec(shp, lambda i: (i,) + (0,) * (len(shp) - 1))
    return pl.pallas_call(
        _proj_body,
        grid=(_GRID,),
        in_specs=[row((_BLK, D_IN)), row((_BLK, D_IN)),
                  full((D_IN, H)), full((D_IN, H)),
                  full((8, H)), full((8, H))],
        out_specs=[row((_BLK, H)), row((_BLK, H))],
        out_shape=[jax.ShapeDtypeStruct((N, H), jnp.float32),
                   jax.ShapeDtypeStruct((N, H), jnp.float32)],
    )(xu, xi, wu, wi, jnp.broadcast_to(bu, (8, H)), jnp.broadcast_to(bi, (8, H)))


_full = lambda shp: pl.BlockSpec(shp, lambda i: (0,) * len(shp))
_row = lambda shp: pl.BlockSpec(shp, lambda i: (i,) + (0,) * (len(shp) - 1))


def _ln_act(n, g, be, relu):
    m = jnp.mean(n, axis=-1, keepdims=True)
    v = jnp.mean((n - m) * (n - m), axis=-1, keepdims=True)
    y = (n - m) * lax.rsqrt(v + 1e-5) * g[0:1, :] + be[0:1, :]
    return jnp.maximum(y, 0.0) if relu else y


def _c1_side(sr, cr, hr, wl, wr, b, g, be, wn, o, op):
    cnt = cr[...][:, 0:1]
    mean = sr[...] / jnp.maximum(cnt, 1.0)
    n = (jnp.dot(mean, wl[...], preferred_element_type=jnp.float32)
         + jnp.dot(hr[...], wr[...], preferred_element_type=jnp.float32)
         + b[0:1, :])
    y = _ln_act(n, g, be, True)
    o[...] = y
    op[...] = jnp.dot(y, wn[...], preferred_element_type=jnp.float32)


def _combine1_body(s_a, c_a, h_a, wl_a, wr_a, b_a, g_a, be_a, wn_a,
                   s_b, c_b, h_b, wl_b, wr_b, b_b, g_b, be_b, wn_b,
                   o_a, op_a, o_b, op_b):
    _c1_side(s_a, c_a, h_a, wl_a, wr_a, b_a, g_a, be_a, wn_a, o_a, op_a)
    _c1_side(s_b, c_b, h_b, wl_b, wr_b, b_b, g_b, be_b, wn_b, o_b, op_b)


def _combine_l1(s_a, c_a, h_a, wl_a, wr_a, b_a, g_a, be_a, wn_a,
                s_b, c_b, h_b, wl_b, wr_b, b_b, g_b, be_b, wn_b):
    bc = lambda x: jnp.broadcast_to(x, (8, H))
    side = [_row((_BLK, H)), _row((_BLK, 16)), _row((_BLK, H)),
            _full((H, H)), _full((H, H)),
            _full((8, H)), _full((8, H)), _full((8, H)), _full((H, OUT))]
    return pl.pallas_call(
        _combine1_body,
        grid=(_GRID,),
        in_specs=side + side,
        out_specs=[_row((_BLK, H)), _row((_BLK, OUT)),
                   _row((_BLK, H)), _row((_BLK, OUT))],
        out_shape=[jax.ShapeDtypeStruct((N, H), jnp.float32),
                   jax.ShapeDtypeStruct((N, OUT), jnp.float32),
                   jax.ShapeDtypeStruct((N, H), jnp.float32),
                   jax.ShapeDtypeStruct((N, OUT), jnp.float32)],
    )(s_a, c_a, h_a, wl_a, wr_a, bc(b_a), bc(g_a), bc(be_a), wn_a,
      s_b, c_b, h_b, wl_b, wr_b, bc(b_b), bc(g_b), bc(be_b), wn_b)


def _c2_side(sr, cr, hr, wr, b, g, be, o):
    cnt = cr[...][:, 0:1]
    n = (sr[...] / jnp.maximum(cnt, 1.0)
         + jnp.dot(hr[...], wr[...], preferred_element_type=jnp.float32)
         + b[0:1, :])
    o[...] = _ln_act(n, g, be, False)


def _combine2_body(s_a, c_a, h_a, wr_a, b_a, g_a, be_a,
                   s_b, c_b, h_b, wr_b, b_b, g_b, be_b, o_a, o_b):
    _c2_side(s_a, c_a, h_a, wr_a, b_a, g_a, be_a, o_a)
    _c2_side(s_b, c_b, h_b, wr_b, b_b, g_b, be_b, o_b)


def _combine_l2(s_a, c_a, h_a, wr_a, b_a, g_a, be_a,
                s_b, c_b, h_b, wr_b, b_b, g_b, be_b):
    bc = lambda x: jnp.broadcast_to(x, (8, OUT))
    side = [_row((_BLK, OUT)), _row((_BLK, 16)), _row((_BLK, H)),
            _full((H, OUT)),
            _full((8, OUT)), _full((8, OUT)), _full((8, OUT))]
    return pl.pallas_call(
        _combine2_body,
        grid=(_GRID,),
        in_specs=side + side,
        out_specs=[_row((_BLK, OUT)), _row((_BLK, OUT))],
        out_shape=[jax.ShapeDtypeStruct((N, OUT), jnp.float32),
                   jax.ShapeDtypeStruct((N, OUT), jnp.float32)],
    )(s_a, c_a, h_a, wr_a, bc(b_a), bc(g_a), bc(be_a),
      s_b, c_b, h_b, wr_b, bc(b_b), bc(g_b), bc(be_b))


def _prep_edges(ei):
    pad = E_PAD - E
    src = jnp.concatenate([ei[0], jnp.zeros((pad,), jnp.int32)])
    dst = jnp.concatenate([ei[1], jnp.full((pad,), N, jnp.int32)])
    return src.reshape(NT, NCH, CH), dst.reshape(NT, NCH, CH)


def kernel(x_user, x_item, edge_index_ui, edge_index_iu, Win_u, bin_u, Win_i,
           bin_i, l1_Wl_ui, l1_Wr_ui, l1_b_ui, l1_Wl_iu, l1_Wr_iu, l1_b_iu,
           l1_ln_g_u, l1_ln_b_u, l1_ln_g_i, l1_ln_b_i, l2_Wl_ui, l2_Wr_ui,
           l2_b_ui, l2_Wl_iu, l2_Wr_iu, l2_b_iu, l2_ln_g_u, l2_ln_b_u,
           l2_ln_g_i, l2_ln_b_i):
    sui, dui = _prep_edges(edge_index_ui)
    siu, diu = _prep_edges(edge_index_iu)

    h_u, h_i = _proj(x_user, x_item, Win_u, bin_u, Win_i, bin_i)

    s_ui, s_iu, c_ui, c_iu = _seg_sum_cnt(h_u, h_i, sui, dui, siu, diu)
    h_i2, hp_i2, h_u2, hp_u2 = _combine_l1(
        s_ui, c_ui, h_i, l1_Wl_ui, l1_Wr_ui, l1_b_ui, l1_ln_g_i, l1_ln_b_i,
        l2_Wl_iu,
        s_iu, c_iu, h_u, l1_Wl_iu, l1_Wr_iu, l1_b_iu, l1_ln_g_u, l1_ln_b_u,
        l2_Wl_ui)

    s2_ui, s2_iu = _seg_sum(hp_u2, hp_i2, sui, dui, siu, diu, OUT)
    out_i, out_u = _combine_l2(
        s2_ui, c_ui, h_i2, l2_Wr_ui, l2_b_ui, l2_ln_g_i, l2_ln_b_i,
        s2_iu, c_iu, h_u2, l2_Wr_iu, l2_b_iu, l2_ln_g_u, l2_ln_b_u)

    return (out_u, out_i)


# speedup vs baseline: 18.3181x; 1.0318x over previous
import jax
import jax.numpy as jnp
from jax import lax
from jax.experimental import pallas as pl
from jax.experimental.pallas import tpu as pltpu
from jax.experimental.pallas import tpu_sc as plsc

N = 10000
E = 300000
D_IN = 128
H = 64
OUT = 32

NT = 16
CH = 128
NCH = 148
NB = 4
E_PAD = NT * NCH * CH
NP = 10240
RPT = NP // NT

_MESH = plsc.VectorSubcoreMesh(core_axis_name="c", subcore_axis_name="s")
_SC_PARAMS = pltpu.CompilerParams(use_tc_tiling_on_sc=False)


def _pipelined_scatter(sv, dv, table, acc, rows, gs, ss,
                       cac=None, ones_v=None, cs=None):
    W = NB // 2

    def gwait(j, k):
        pltpu.make_async_copy(table.at[sv.at[j]], rows[k], gs[k]).wait()

    def swait(k):
        pltpu.make_async_copy(rows[k], acc.at[dv.at[0]], ss[k]).wait()

    def cwait(k):
        pltpu.make_async_copy(ones_v, cac.at[dv.at[0]], cs[k]).wait()

    for k in range(W):
        pltpu.async_copy(table.at[sv.at[k]], rows[k], gs[k])

    def group(g, carry):
        j0 = g * NB
        for k in range(NB):
            j = j0 + k
            gwait(j, k)
            pltpu.async_copy(rows[k], acc.at[dv.at[j]], ss[k], add=True)
            if cac is not None:
                pltpu.async_copy(ones_v, cac.at[dv.at[j]], cs[k], add=True)
            kn = (k + W) % NB

            @pl.when(j + W < NCH)
            def _(j=j, kn=kn):
                @pl.when(j >= W)
                def _():
                    swait(kn)
                    if cac is not None:
                        cwait(kn)
                pltpu.async_copy(table.at[sv.at[j + W]], rows[kn], gs[kn])
        return carry

    lax.fori_loop(0, NCH // NB, group, 0)
    for k in range(NB):
        swait(k)
        if cac is not None:
            cwait(k)


def _seg_body_cnt(hu, hi, sui, dui, siu, diu, zrow, z16, ones16,
                  o_sui, o_siu, o_cui, o_ciu,
                  acc, cac, sv, dv, r0b, r1b, r2b, r3b, ones_v,
                  g0, g1, g2, g3, s0, s1, s2, s3, c0, c1, c2, c3):
    c = lax.axis_index("c")
    s = lax.axis_index("s")
    r0 = s * RPT
    rows = (r0b, r1b, r2b, r3b)
    gs = (g0, g1, g2, g3)
    ss = (s0, s1, s2, s3)
    cs = (c0, c1, c2, c3)
    pltpu.sync_copy(zrow, acc.at[pl.ds(r0, RPT)])
    pltpu.sync_copy(z16, cac.at[pl.ds(r0, RPT)])
    pltpu.sync_copy(ones16, ones_v)

    def work(src_hbm, dst_hbm, table, o_s, o_c):
        pltpu.sync_copy(src_hbm.at[s], sv)
        pltpu.sync_copy(dst_hbm.at[s], dv)
        plsc.subcore_barrier()
        _pipelined_scatter(sv, dv, table, acc, rows, gs, ss,
                           cac=cac, ones_v=ones_v, cs=cs)
        plsc.subcore_barrier()
        pltpu.sync_copy(acc.at[pl.ds(r0, RPT)], o_s.at[pl.ds(r0, RPT)])
        pltpu.sync_copy(cac.at[pl.ds(r0, RPT)], o_c.at[pl.ds(r0, RPT)])

    @pl.when(c == 0)
    def _():
        work(sui, dui, hu, o_sui, o_cui)

    @pl.when(c == 1)
    def _():
        work(siu, diu, hi, o_siu, o_ciu)


def _seg_body(hu, hi, sui, dui, siu, diu, zrow,
              o_sui, o_siu,
              acc, sv, dv, r0b, r1b, r2b, r3b,
              g0, g1, g2, g3, s0, s1, s2, s3):
    c = lax.axis_index("c")
    s = lax.axis_index("s")
    r0 = s * RPT
    rows = (r0b, r1b, r2b, r3b)
    gs = (g0, g1, g2, g3)
    ss = (s0, s1, s2, s3)
    pltpu.sync_copy(zrow, acc.at[pl.ds(r0, RPT)])

    def work(src_hbm, dst_hbm, table, o_s):
        pltpu.sync_copy(src_hbm.at[s], sv)
        pltpu.sync_copy(dst_hbm.at[s], dv)
        plsc.subcore_barrier()
        _pipelined_scatter(sv, dv, table, acc, rows, gs, ss)
        plsc.subcore_barrier()
        pltpu.sync_copy(acc.at[pl.ds(r0, RPT)], o_s.at[pl.ds(r0, RPT)])

    @pl.when(c == 0)
    def _():
        work(sui, dui, hu, o_sui)

    @pl.when(c == 1)
    def _():
        work(siu, diu, hi, o_siu)


def _seg_sum_cnt(hu, hi, sui, dui, siu, diu):
    zrow = jnp.zeros((RPT, H), jnp.float32)
    z16 = jnp.zeros((RPT, 16), jnp.float32)
    ones16 = jnp.ones((CH, 16), jnp.float32)
    f = pl.kernel(
        _seg_body_cnt,
        out_type=[
            jax.ShapeDtypeStruct((NP, H), jnp.float32),
            jax.ShapeDtypeStruct((NP, H), jnp.float32),
            jax.ShapeDtypeStruct((NP, 16), jnp.float32),
            jax.ShapeDtypeStruct((NP, 16), jnp.float32),
        ],
        mesh=_MESH,
        compiler_params=_SC_PARAMS,
        scratch_types=[
            pltpu.VMEM_SHARED((NP, H), jnp.float32),
            pltpu.VMEM_SHARED((NP, 16), jnp.float32),
            pltpu.VMEM((NCH, CH), jnp.int32),
            pltpu.VMEM((NCH, CH), jnp.int32),
        ] + [pltpu.VMEM((CH, H), jnp.float32)] * NB + [
            pltpu.VMEM((CH, 16), jnp.float32),
        ] + [pltpu.SemaphoreType.DMA] * (3 * NB),
    )
    return f(hu, hi, sui, dui, siu, diu, zrow, z16, ones16)


def _seg_sum(hu, hi, sui, dui, siu, diu, feat):
    zrow = jnp.zeros((RPT, feat), jnp.float32)
    f = pl.kernel(
        _seg_body,
        out_type=[
            jax.ShapeDtypeStruct((NP, feat), jnp.float32),
            jax.ShapeDtypeStruct((NP, feat), jnp.float32),
        ],
        mesh=_MESH,
        compiler_params=_SC_PARAMS,
        scratch_types=[
            pltpu.VMEM_SHARED((NP, feat), jnp.float32),
            pltpu.VMEM((NCH, CH), jnp.int32),
            pltpu.VMEM((NCH, CH), jnp.int32),
        ] + [pltpu.VMEM((CH, feat), jnp.float32)] * NB
          + [pltpu.SemaphoreType.DMA] * (2 * NB),
    )
    return f(hu, hi, sui, dui, siu, diu, zrow)


_BLK = 2000
_GRID = N // _BLK


def _proj_body(xu, xi, wu, wi, bu, bi, ou, oi):
    ou[...] = jnp.maximum(
        jnp.dot(xu[...], wu[...], preferred_element_type=jnp.float32)
        + bu[0:1, :], 0.0)
    oi[...] = jnp.maximum(
        jnp.dot(xi[...], wi[...], preferred_element_type=jnp.float32)
        + bi[0:1, :], 0.0)


def _proj(xu, xi, wu, bu, wi, bi):
    full = lambda shp: pl.BlockSpec(shp, lambda i: (0,) * len(shp))
    row = lambda shp: pl.BlockSpec(shp, lambda i: (i,) + (0,) * (len(shp) - 1))
    return pl.pallas_call(
        _proj_body,
        grid=(_GRID,),
        in_specs=[row((_BLK, D_IN)), row((_BLK, D_IN)),
                  full((D_IN, H)), full((D_IN, H)),
                  full((1, H)), full((1, H))],
        out_specs=[row((_BLK, H)), row((_BLK, H))],
        out_shape=[jax.ShapeDtypeStruct((N, H), jnp.float32),
                   jax.ShapeDtypeStruct((N, H), jnp.float32)],
    )(xu, xi, wu, wi, bu.reshape(1, H), bi.reshape(1, H))


_full = lambda shp: pl.BlockSpec(shp, lambda i: (0,) * len(shp))
_row = lambda shp: pl.BlockSpec(shp, lambda i: (i,) + (0,) * (len(shp) - 1))


def _ln_act(n, g, be, relu):
    m = jnp.mean(n, axis=-1, keepdims=True)
    v = jnp.mean((n - m) * (n - m), axis=-1, keepdims=True)
    y = (n - m) * lax.rsqrt(v + 1e-5) * g[0:1, :] + be[0:1, :]
    return jnp.maximum(y, 0.0) if relu else y


def _c1_side(sr, cr, hr, wl, wr, b, g, be, wn, o, op):
    cnt = cr[...][:, 0:1]
    mean = sr[...] / jnp.maximum(cnt, 1.0)
    n = (jnp.dot(mean, wl[...], preferred_element_type=jnp.float32)
         + jnp.dot(hr[...], wr[...], preferred_element_type=jnp.float32)
         + b[0:1, :])
    y = _ln_act(n, g, be, True)
    o[...] = y
    op[...] = jnp.dot(y, wn[...], preferred_element_type=jnp.float32)


def _combine1_body(s_a, c_a, h_a, wl_a, wr_a, b_a, g_a, be_a, wn_a,
                   s_b, c_b, h_b, wl_b, wr_b, b_b, g_b, be_b, wn_b,
                   o_a, op_a, o_b, op_b):
    _c1_side(s_a, c_a, h_a, wl_a, wr_a, b_a, g_a, be_a, wn_a, o_a, op_a)
    _c1_side(s_b, c_b, h_b, wl_b, wr_b, b_b, g_b, be_b, wn_b, o_b, op_b)


def _combine_l1(s_a, c_a, h_a, wl_a, wr_a, b_a, g_a, be_a, wn_a,
                s_b, c_b, h_b, wl_b, wr_b, b_b, g_b, be_b, wn_b):
    bc = lambda x: x.reshape(1, H)
    side = [_row((_BLK, H)), _row((_BLK, 16)), _row((_BLK, H)),
            _full((H, H)), _full((H, H)),
            _full((1, H)), _full((1, H)), _full((1, H)), _full((H, OUT))]
    return pl.pallas_call(
        _combine1_body,
        grid=(_GRID,),
        in_specs=side + side,
        out_specs=[_row((_BLK, H)), _row((_BLK, OUT)),
                   _row((_BLK, H)), _row((_BLK, OUT))],
        out_shape=[jax.ShapeDtypeStruct((N, H), jnp.float32),
                   jax.ShapeDtypeStruct((N, OUT), jnp.float32),
                   jax.ShapeDtypeStruct((N, H), jnp.float32),
                   jax.ShapeDtypeStruct((N, OUT), jnp.float32)],
    )(s_a, c_a, h_a, wl_a, wr_a, bc(b_a), bc(g_a), bc(be_a), wn_a,
      s_b, c_b, h_b, wl_b, wr_b, bc(b_b), bc(g_b), bc(be_b), wn_b)


def _c2_side(sr, cr, hr, wr, b, g, be, o):
    cnt = cr[...][:, 0:1]
    n = (sr[...] / jnp.maximum(cnt, 1.0)
         + jnp.dot(hr[...], wr[...], preferred_element_type=jnp.float32)
         + b[0:1, :])
    o[...] = _ln_act(n, g, be, False)


def _combine2_body(s_a, c_a, h_a, wr_a, b_a, g_a, be_a,
                   s_b, c_b, h_b, wr_b, b_b, g_b, be_b, o_a, o_b):
    _c2_side(s_a, c_a, h_a, wr_a, b_a, g_a, be_a, o_a)
    _c2_side(s_b, c_b, h_b, wr_b, b_b, g_b, be_b, o_b)


def _combine_l2(s_a, c_a, h_a, wr_a, b_a, g_a, be_a,
                s_b, c_b, h_b, wr_b, b_b, g_b, be_b):
    bc = lambda x: x.reshape(1, OUT)
    side = [_row((_BLK, OUT)), _row((_BLK, 16)), _row((_BLK, H)),
            _full((H, OUT)),
            _full((1, OUT)), _full((1, OUT)), _full((1, OUT))]
    return pl.pallas_call(
        _combine2_body,
        grid=(_GRID,),
        in_specs=side + side,
        out_specs=[_row((_BLK, OUT)), _row((_BLK, OUT))],
        out_shape=[jax.ShapeDtypeStruct((N, OUT), jnp.float32),
                   jax.ShapeDtypeStruct((N, OUT), jnp.float32)],
    )(s_a, c_a, h_a, wr_a, bc(b_a), bc(g_a), bc(be_a),
      s_b, c_b, h_b, wr_b, bc(b_b), bc(g_b), bc(be_b))


def _prep_edges(ei):
    pad = E_PAD - E
    src = jnp.concatenate([ei[0], jnp.zeros((pad,), jnp.int32)])
    dst = jnp.concatenate([ei[1], jnp.full((pad,), N, jnp.int32)])
    return src.reshape(NT, NCH, CH), dst.reshape(NT, NCH, CH)


def kernel(x_user, x_item, edge_index_ui, edge_index_iu, Win_u, bin_u, Win_i,
           bin_i, l1_Wl_ui, l1_Wr_ui, l1_b_ui, l1_Wl_iu, l1_Wr_iu, l1_b_iu,
           l1_ln_g_u, l1_ln_b_u, l1_ln_g_i, l1_ln_b_i, l2_Wl_ui, l2_Wr_ui,
           l2_b_ui, l2_Wl_iu, l2_Wr_iu, l2_b_iu, l2_ln_g_u, l2_ln_b_u,
           l2_ln_g_i, l2_ln_b_i):
    sui, dui = _prep_edges(edge_index_ui)
    siu, diu = _prep_edges(edge_index_iu)

    h_u, h_i = _proj(x_user, x_item, Win_u, bin_u, Win_i, bin_i)

    s_ui, s_iu, c_ui, c_iu = _seg_sum_cnt(h_u, h_i, sui, dui, siu, diu)
    h_i2, hp_i2, h_u2, hp_u2 = _combine_l1(
        s_ui, c_ui, h_i, l1_Wl_ui, l1_Wr_ui, l1_b_ui, l1_ln_g_i, l1_ln_b_i,
        l2_Wl_iu,
        s_iu, c_iu, h_u, l1_Wl_iu, l1_Wr_iu, l1_b_iu, l1_ln_g_u, l1_ln_b_u,
        l2_Wl_ui)

    s2_ui, s2_iu = _seg_sum(hp_u2, hp_i2, sui, dui, siu, diu, OUT)
    out_i, out_u = _combine_l2(
        s2_ui, c_ui, h_i2, l2_Wr_ui, l2_b_ui, l2_ln_g_i, l2_ln_b_i,
        s2_iu, c_iu, h_u2, l2_Wr_iu, l2_b_iu, l2_ln_g_u, l2_ln_b_u)

    return (out_u, out_i)


# speedup vs baseline: 20.3963x; 1.1135x over previous
import jax
import jax.numpy as jnp
from jax import lax
from jax.experimental import pallas as pl
from jax.experimental.pallas import tpu as pltpu
from jax.experimental.pallas import tpu_sc as plsc

N = 10000
E = 300000
D_IN = 128
H = 64
OUT = 32

NT = 16
CH = 128
NCH = 148
NB = 4
E_PAD = NT * NCH * CH
NP = 10240
RPT = NP // NT

_MESH = plsc.VectorSubcoreMesh(core_axis_name="c", subcore_axis_name="s")
_SC_PARAMS = pltpu.CompilerParams(use_tc_tiling_on_sc=False)


def _pipelined_scatter(sv, dv, table, acc, rows, gs, ss,
                       cac=None, ones_v=None, cs=None):
    W = NB // 2

    def gwait(j, k):
        pltpu.make_async_copy(table.at[sv.at[j]], rows[k], gs[k]).wait()

    def swait(k):
        pltpu.make_async_copy(rows[k], acc.at[dv.at[0]], ss[k]).wait()

    def cwait(k):
        pltpu.make_async_copy(ones_v, cac.at[dv.at[0]], cs[k]).wait()

    for k in range(W):
        pltpu.async_copy(table.at[sv.at[k]], rows[k], gs[k])

    def group(g, carry):
        j0 = g * NB
        for k in range(NB):
            j = j0 + k
            gwait(j, k)
            pltpu.async_copy(rows[k], acc.at[dv.at[j]], ss[k], add=True)
            if cac is not None:
                pltpu.async_copy(ones_v, cac.at[dv.at[j]], cs[k], add=True)
            kn = (k + W) % NB

            @pl.when(j + W < NCH)
            def _(j=j, kn=kn):
                @pl.when(j >= W)
                def _():
                    swait(kn)
                    if cac is not None:
                        cwait(kn)
                pltpu.async_copy(table.at[sv.at[j + W]], rows[kn], gs[kn])
        return carry

    lax.fori_loop(0, NCH // NB, group, 0)
    for k in range(NB):
        swait(k)
        if cac is not None:
            cwait(k)


def _seg_body_cnt(hu, hi, sui, dui, siu, diu, zrow, z16, ones16,
                  o_sui, o_siu, o_cui, o_ciu,
                  acc, cac, sv, dv, r0b, r1b, r2b, r3b, ones_v,
                  g0, g1, g2, g3, s0, s1, s2, s3, c0, c1, c2, c3):
    c = lax.axis_index("c")
    s = lax.axis_index("s")
    r0 = s * RPT
    rows = (r0b, r1b, r2b, r3b)
    gs = (g0, g1, g2, g3)
    ss = (s0, s1, s2, s3)
    cs = (c0, c1, c2, c3)
    pltpu.sync_copy(zrow, acc.at[pl.ds(r0, RPT)])
    pltpu.sync_copy(z16, cac.at[pl.ds(r0, RPT)])
    pltpu.sync_copy(ones16, ones_v)

    def work(src_hbm, dst_hbm, table, o_s, o_c):
        pltpu.sync_copy(src_hbm.at[s], sv)
        pltpu.sync_copy(dst_hbm.at[s], dv)
        plsc.subcore_barrier()
        _pipelined_scatter(sv, dv, table, acc, rows, gs, ss,
                           cac=cac, ones_v=ones_v, cs=cs)
        plsc.subcore_barrier()
        pltpu.sync_copy(acc.at[pl.ds(r0, RPT)], o_s.at[pl.ds(r0, RPT)])
        pltpu.sync_copy(cac.at[pl.ds(r0, RPT)], o_c.at[pl.ds(r0, RPT)])

    @pl.when(c == 0)
    def _():
        work(sui, dui, hu, o_sui, o_cui)

    @pl.when(c == 1)
    def _():
        work(siu, diu, hi, o_siu, o_ciu)


def _seg_body(hu, hi, sui, dui, siu, diu, zrow,
              o_sui, o_siu,
              acc, tab, sv, dv, r0b, r1b, r2b, r3b,
              g0, g1, g2, g3, s0, s1, s2, s3):
    c = lax.axis_index("c")
    s = lax.axis_index("s")
    r0 = s * RPT
    rows = (r0b, r1b, r2b, r3b)
    gs = (g0, g1, g2, g3)
    ss = (s0, s1, s2, s3)
    pltpu.sync_copy(zrow, acc.at[pl.ds(r0, RPT)])

    def work(src_hbm, dst_hbm, table_hbm, o_s):
        pltpu.sync_copy(table_hbm.at[pl.ds(r0, RPT)], tab.at[pl.ds(r0, RPT)])
        pltpu.sync_copy(src_hbm.at[s], sv)
        pltpu.sync_copy(dst_hbm.at[s], dv)
        plsc.subcore_barrier()
        _pipelined_scatter(sv, dv, tab, acc, rows, gs, ss)
        plsc.subcore_barrier()
        pltpu.sync_copy(acc.at[pl.ds(r0, RPT)], o_s.at[pl.ds(r0, RPT)])

    @pl.when(c == 0)
    def _():
        work(sui, dui, hu, o_sui)

    @pl.when(c == 1)
    def _():
        work(siu, diu, hi, o_siu)


def _seg_sum_cnt(hu, hi, sui, dui, siu, diu):
    zrow = jnp.zeros((RPT, H), jnp.float32)
    z16 = jnp.zeros((RPT, 16), jnp.float32)
    ones16 = jnp.ones((CH, 16), jnp.float32)
    f = pl.kernel(
        _seg_body_cnt,
        out_type=[
            jax.ShapeDtypeStruct((NP, H), jnp.float32),
            jax.ShapeDtypeStruct((NP, H), jnp.float32),
            jax.ShapeDtypeStruct((NP, 16), jnp.float32),
            jax.ShapeDtypeStruct((NP, 16), jnp.float32),
        ],
        mesh=_MESH,
        compiler_params=_SC_PARAMS,
        scratch_types=[
            pltpu.VMEM_SHARED((NP, H), jnp.float32),
            pltpu.VMEM_SHARED((NP, 16), jnp.float32),
            pltpu.VMEM((NCH, CH), jnp.int32),
            pltpu.VMEM((NCH, CH), jnp.int32),
        ] + [pltpu.VMEM((CH, H), jnp.float32)] * NB + [
            pltpu.VMEM((CH, 16), jnp.float32),
        ] + [pltpu.SemaphoreType.DMA] * (3 * NB),
    )
    return f(hu, hi, sui, dui, siu, diu, zrow, z16, ones16)


def _seg_sum(hu, hi, sui, dui, siu, diu, feat):
    zrow = jnp.zeros((RPT, feat), jnp.float32)
    f = pl.kernel(
        _seg_body,
        out_type=[
            jax.ShapeDtypeStruct((NP, feat), jnp.float32),
            jax.ShapeDtypeStruct((NP, feat), jnp.float32),
        ],
        mesh=_MESH,
        compiler_params=_SC_PARAMS,
        scratch_types=[
            pltpu.VMEM_SHARED((NP, feat), jnp.float32),
            pltpu.VMEM_SHARED((NP, feat), jnp.float32),
            pltpu.VMEM((NCH, CH), jnp.int32),
            pltpu.VMEM((NCH, CH), jnp.int32),
        ] + [pltpu.VMEM((CH, feat), jnp.float32)] * NB
          + [pltpu.SemaphoreType.DMA] * (2 * NB),
    )
    return f(hu, hi, sui, dui, siu, diu, zrow)


_BLK = 2000
_GRID = N // _BLK


def _proj_body(xu, xi, wu, wi, bu, bi, ou, oi):
    ou[...] = jnp.maximum(
        jnp.dot(xu[...], wu[...], preferred_element_type=jnp.float32)
        + bu[0:1, :], 0.0)
    oi[...] = jnp.maximum(
        jnp.dot(xi[...], wi[...], preferred_element_type=jnp.float32)
        + bi[0:1, :], 0.0)


def _proj(xu, xi, wu, bu, wi, bi):
    full = lambda shp: pl.BlockSpec(shp, lambda i: (0,) * len(shp))
    row = lambda shp: pl.BlockSpec(shp, lambda i: (i,) + (0,) * (len(shp) - 1))
    return pl.pallas_call(
        _proj_body,
        grid=(_GRID,),
        in_specs=[row((_BLK, D_IN)), row((_BLK, D_IN)),
                  full((D_IN, H)), full((D_IN, H)),
                  full((1, H)), full((1, H))],
        out_specs=[row((_BLK, H)), row((_BLK, H))],
        out_shape=[jax.ShapeDtypeStruct((N, H), jnp.float32),
                   jax.ShapeDtypeStruct((N, H), jnp.float32)],
    )(xu, xi, wu, wi, bu.reshape(1, H), bi.reshape(1, H))


_full = lambda shp: pl.BlockSpec(shp, lambda i: (0,) * len(shp))
_row = lambda shp: pl.BlockSpec(shp, lambda i: (i,) + (0,) * (len(shp) - 1))


def _ln_act(n, g, be, relu):
    m = jnp.mean(n, axis=-1, keepdims=True)
    v = jnp.mean((n - m) * (n - m), axis=-1, keepdims=True)
    y = (n - m) * lax.rsqrt(v + 1e-5) * g[0:1, :] + be[0:1, :]
    return jnp.maximum(y, 0.0) if relu else y


def _c1_side(sr, cr, hr, wl, wr, b, g, be, wn, o, op):
    cnt = cr[...][:, 0:1]
    mean = sr[...] / jnp.maximum(cnt, 1.0)
    n = (jnp.dot(mean, wl[...], preferred_element_type=jnp.float32)
         + jnp.dot(hr[...], wr[...], preferred_element_type=jnp.float32)
         + b[0:1, :])
    y = _ln_act(n, g, be, True)
    o[...] = y
    op[...] = jnp.dot(y, wn[...], preferred_element_type=jnp.float32)


def _combine1_body(s_a, c_a, h_a, wl_a, wr_a, b_a, g_a, be_a, wn_a,
                   s_b, c_b, h_b, wl_b, wr_b, b_b, g_b, be_b, wn_b,
                   o_a, op_a, o_b, op_b):
    _c1_side(s_a, c_a, h_a, wl_a, wr_a, b_a, g_a, be_a, wn_a, o_a, op_a)
    _c1_side(s_b, c_b, h_b, wl_b, wr_b, b_b, g_b, be_b, wn_b, o_b, op_b)


def _combine_l1(s_a, c_a, h_a, wl_a, wr_a, b_a, g_a, be_a, wn_a,
                s_b, c_b, h_b, wl_b, wr_b, b_b, g_b, be_b, wn_b):
    bc = lambda x: x.reshape(1, H)
    side = [_row((_BLK, H)), _row((_BLK, 16)), _row((_BLK, H)),
            _full((H, H)), _full((H, H)),
            _full((1, H)), _full((1, H)), _full((1, H)), _full((H, OUT))]
    return pl.pallas_call(
        _combine1_body,
        grid=(_GRID,),
        in_specs=side + side,
        out_specs=[_row((_BLK, H)), _row((_BLK, OUT)),
                   _row((_BLK, H)), _row((_BLK, OUT))],
        out_shape=[jax.ShapeDtypeStruct((N, H), jnp.float32),
                   jax.ShapeDtypeStruct((NP, OUT), jnp.float32),
                   jax.ShapeDtypeStruct((N, H), jnp.float32),
                   jax.ShapeDtypeStruct((NP, OUT), jnp.float32)],
    )(s_a, c_a, h_a, wl_a, wr_a, bc(b_a), bc(g_a), bc(be_a), wn_a,
      s_b, c_b, h_b, wl_b, wr_b, bc(b_b), bc(g_b), bc(be_b), wn_b)


def _c2_side(sr, cr, hr, wr, b, g, be, o):
    cnt = cr[...][:, 0:1]
    n = (sr[...] / jnp.maximum(cnt, 1.0)
         + jnp.dot(hr[...], wr[...], preferred_element_type=jnp.float32)
         + b[0:1, :])
    o[...] = _ln_act(n, g, be, False)


def _combine2_body(s_a, c_a, h_a, wr_a, b_a, g_a, be_a,
                   s_b, c_b, h_b, wr_b, b_b, g_b, be_b, o_a, o_b):
    _c2_side(s_a, c_a, h_a, wr_a, b_a, g_a, be_a, o_a)
    _c2_side(s_b, c_b, h_b, wr_b, b_b, g_b, be_b, o_b)


def _combine_l2(s_a, c_a, h_a, wr_a, b_a, g_a, be_a,
                s_b, c_b, h_b, wr_b, b_b, g_b, be_b):
    bc = lambda x: x.reshape(1, OUT)
    side = [_row((_BLK, OUT)), _row((_BLK, 16)), _row((_BLK, H)),
            _full((H, OUT)),
            _full((1, OUT)), _full((1, OUT)), _full((1, OUT))]
    return pl.pallas_call(
        _combine2_body,
        grid=(_GRID,),
        in_specs=side + side,
        out_specs=[_row((_BLK, OUT)), _row((_BLK, OUT))],
        out_shape=[jax.ShapeDtypeStruct((N, OUT), jnp.float32),
                   jax.ShapeDtypeStruct((N, OUT), jnp.float32)],
    )(s_a, c_a, h_a, wr_a, bc(b_a), bc(g_a), bc(be_a),
      s_b, c_b, h_b, wr_b, bc(b_b), bc(g_b), bc(be_b))


def _prep_edges(ei):
    pad = E_PAD - E
    src = jnp.concatenate([ei[0], jnp.zeros((pad,), jnp.int32)])
    dst = jnp.concatenate([ei[1], jnp.full((pad,), N, jnp.int32)])
    return src.reshape(NT, NCH, CH), dst.reshape(NT, NCH, CH)


def kernel(x_user, x_item, edge_index_ui, edge_index_iu, Win_u, bin_u, Win_i,
           bin_i, l1_Wl_ui, l1_Wr_ui, l1_b_ui, l1_Wl_iu, l1_Wr_iu, l1_b_iu,
           l1_ln_g_u, l1_ln_b_u, l1_ln_g_i, l1_ln_b_i, l2_Wl_ui, l2_Wr_ui,
           l2_b_ui, l2_Wl_iu, l2_Wr_iu, l2_b_iu, l2_ln_g_u, l2_ln_b_u,
           l2_ln_g_i, l2_ln_b_i):
    sui, dui = _prep_edges(edge_index_ui)
    siu, diu = _prep_edges(edge_index_iu)

    h_u, h_i = _proj(x_user, x_item, Win_u, bin_u, Win_i, bin_i)

    s_ui, s_iu, c_ui, c_iu = _seg_sum_cnt(h_u, h_i, sui, dui, siu, diu)
    h_i2, hp_i2, h_u2, hp_u2 = _combine_l1(
        s_ui, c_ui, h_i, l1_Wl_ui, l1_Wr_ui, l1_b_ui, l1_ln_g_i, l1_ln_b_i,
        l2_Wl_iu,
        s_iu, c_iu, h_u, l1_Wl_iu, l1_Wr_iu, l1_b_iu, l1_ln_g_u, l1_ln_b_u,
        l2_Wl_ui)

    s2_ui, s2_iu = _seg_sum(hp_u2, hp_i2, sui, dui, siu, diu, OUT)
    out_i, out_u = _combine_l2(
        s2_ui, c_ui, h_i2, l2_Wr_ui, l2_b_ui, l2_ln_g_i, l2_ln_b_i,
        s2_iu, c_iu, h_u2, l2_Wr_iu, l2_b_iu, l2_ln_g_u, l2_ln_b_u)

    return (out_u, out_i)
